# Initial kernel scaffold; baseline (speedup 1.0000x reference)
import jax
import jax.numpy as jnp
from jax.experimental import pallas as pl


def kernel(h, edge_index, e, W_emb, b_emb, W0, a0, g0, bta0, W1, a1, g1, bta1, Wf, af, gf, btf, Watt, batt, Wc, bc, Wo, bo, Wco, bco):
    raise NotImplementedError("write your pallas kernel here")


# speedup vs baseline: 4.5456x; 4.5456x over previous
import functools

import jax
import jax.numpy as jnp
from jax.experimental import pallas as pl
from jax.experimental.pallas import tpu as pltpu

_N = 10000
_E = 160000
_ROWS = 1000
_NB = _N // _ROWS


def _mm_bias_body(x_ref, w_ref, b_ref, o_ref):
    o_ref[...] = jnp.dot(x_ref[...], w_ref[...],
                         preferred_element_type=jnp.float32) + b_ref[...]


def _emb_matmul(h, W, b2):
    k = h.shape[1]
    m = W.shape[1]
    return pl.pallas_call(
        _mm_bias_body,
        grid=(_NB,),
        in_specs=[
            pl.BlockSpec((_ROWS, k), lambda i: (i, 0)),
            pl.BlockSpec((k, m), lambda i: (0, 0)),
            pl.BlockSpec((1, m), lambda i: (0, 0)),
        ],
        out_specs=pl.BlockSpec((_ROWS, m), lambda i: (i, 0)),
        out_shape=jax.ShapeDtypeStruct((_N, m), jnp.float32),
    )(h, W, b2)


def _proj_body(x_ref, wcat_ref, wsd_ref, z0, z1, z2, z3, ss_ref, sd_ref):
    x = x_ref[...]
    z = jnp.dot(x, wcat_ref[...], preferred_element_type=jnp.float32)
    s = jnp.dot(x, wsd_ref[...], preferred_element_type=jnp.float32)
    z0[...] = z[:, 0:128]
    z1[...] = z[:, 128:256]
    z2[...] = z[:, 256:384]
    z3[...] = z[:, 384:512]
    ss_ref[...] = s[:, 0:16]
    sd_ref[...] = s[:, 16:32]


def _proj(x, Wcat, Wsd):
    outs = (
        [jax.ShapeDtypeStruct((_N, 128), jnp.float32) for _ in range(4)]
        + [jax.ShapeDtypeStruct((_N, 16), jnp.float32) for _ in range(2)]
    )
    out_specs = (
        [pl.BlockSpec((_ROWS, 128), lambda i: (i, 0)) for _ in range(4)]
        + [pl.BlockSpec((_ROWS, 16), lambda i: (i, 0)) for _ in range(2)]
    )
    return pl.pallas_call(
        _proj_body,
        grid=(_NB,),
        in_specs=[
            pl.BlockSpec((_ROWS, 512), lambda i: (i, 0)),
            pl.BlockSpec((512, 512), lambda i: (0, 0)),
            pl.BlockSpec((512, 32), lambda i: (0, 0)),
        ],
        out_specs=out_specs,
        out_shape=outs,
    )(x, Wcat, Wsd)


def _t_block(u_refs, d0_ref, d1_ref, nheads):
    d = d0_ref[...] + d1_ref[...]
    cols = []
    if nheads == 1:
        dh = d[:, 0:1] + 1e-9
        for u in u_refs:
            cols.append(u[...] / dh)
    else:
        for j, u in enumerate(u_refs):
            ub = u[...]
            h0 = 2 * j
            cols.append(ub[:, 0:64] / (d[:, h0:h0 + 1] + 1e-9))
            cols.append(ub[:, 64:128] / (d[:, h0 + 1:h0 + 2] + 1e-9))
    return jnp.concatenate(cols, axis=1)


def _stats_body(u0, u1, u2, u3, d0_ref, d1_ref, o_ref, *, nheads):
    i = pl.program_id(0)

    @pl.when(i == 0)
    def _():
        o_ref[...] = jnp.zeros_like(o_ref)

    t = _t_block((u0, u1, u2, u3), d0_ref, d1_ref, nheads)
    s = jnp.sum(t, axis=0, keepdims=True)
    ss = jnp.sum(t * t, axis=0, keepdims=True)
    o_ref[...] += jnp.concatenate([s, ss], axis=0)


def _stats(u, d0, d1, nheads):
    return pl.pallas_call(
        functools.partial(_stats_body, nheads=nheads),
        grid=(_NB,),
        in_specs=[pl.BlockSpec((_ROWS, 128), lambda i: (i, 0)) for _ in range(4)]
        + [pl.BlockSpec((_ROWS, 16), lambda i: (i, 0)) for _ in range(2)],
        out_specs=pl.BlockSpec((2, 512), lambda i: (0, 0)),
        out_shape=jax.ShapeDtypeStruct((2, 512), jnp.float32),
    )(*u, d0, d1)


def _apply_proj_body(u0, u1, u2, u3, d0_ref, d1_ref, sums_ref, g_ref, b_ref,
                     res_ref, wcat_ref, wsd_ref,
                     y_ref, z0, z1, z2, z3, ss_ref, sd_ref, *, nheads,
                     do_proj):
    t = _t_block((u0, u1, u2, u3), d0_ref, d1_ref, nheads)
    sums = sums_ref[...]
    mean = sums[0:1] * (1.0 / _N)
    var = sums[1:2] * (1.0 / _N) - mean * mean
    rstd = jax.lax.rsqrt(var + 1e-5)
    yv = (t - mean) * rstd * g_ref[...] + b_ref[...]
    yv = jnp.where(yv > 0, yv, jnp.exp(jnp.minimum(yv, 0.0)) - 1.0)
    yv = yv + res_ref[...]
    y_ref[...] = yv
    if do_proj:
        z = jnp.dot(yv, wcat_ref[...], preferred_element_type=jnp.float32)
        s = jnp.dot(yv, wsd_ref[...], preferred_element_type=jnp.float32)
        z0[...] = z[:, 0:128]
        z1[...] = z[:, 128:256]
        z2[...] = z[:, 256:384]
        z3[...] = z[:, 384:512]
        ss_ref[...] = s[:, 0:16]
        sd_ref[...] = s[:, 16:32]


def _apply_proj(u, d0, d1, sums, g2, b2, res, Wcat, Wsd, nheads, do_proj):
    outs = [jax.ShapeDtypeStruct((_N, 512), jnp.float32)]
    out_specs = [pl.BlockSpec((_ROWS, 512), lambda i: (i, 0))]
    if do_proj:
        outs += [jax.ShapeDtypeStruct((_N, 128), jnp.float32) for _ in range(4)]
        outs += [jax.ShapeDtypeStruct((_N, 16), jnp.float32) for _ in range(2)]
        out_specs += [pl.BlockSpec((_ROWS, 128), lambda i: (i, 0))
                      for _ in range(4)]
        out_specs += [pl.BlockSpec((_ROWS, 16), lambda i: (i, 0))
                      for _ in range(2)]
    else:
        outs += [jax.ShapeDtypeStruct((8, 128), jnp.float32)
                 for _ in range(6)]
        out_specs += [pl.BlockSpec((8, 128), lambda i: (0, 0))
                      for _ in range(6)]
    return pl.pallas_call(
        functools.partial(_apply_proj_body, nheads=nheads, do_proj=do_proj),
        grid=(_NB,),
        in_specs=[pl.BlockSpec((_ROWS, 128), lambda i: (i, 0)) for _ in range(4)]
        + [pl.BlockSpec((_ROWS, 16), lambda i: (i, 0)) for _ in range(2)]
        + [
            pl.BlockSpec((2, 512), lambda i: (0, 0)),
            pl.BlockSpec((1, 512), lambda i: (0, 0)),
            pl.BlockSpec((1, 512), lambda i: (0, 0)),
            pl.BlockSpec((_ROWS, 512), lambda i: (i, 0)),
            pl.BlockSpec((512, 512), lambda i: (0, 0)),
            pl.BlockSpec((512, 32), lambda i: (0, 0)),
        ],
        out_specs=out_specs,
        out_shape=outs,
    )(*u, d0, d1, sums, g2, b2, res, Wcat, Wsd)


def _readout_body(y_ref, watt_ref, batt_ref, o_ref):
    i = pl.program_id(0)

    @pl.when(i == 0)
    def _():
        o_ref[...] = jnp.zeros_like(o_ref)

    y = y_ref[...]
    l = jnp.dot(y, watt_ref[...], preferred_element_type=jnp.float32) \
        + batt_ref[...]
    m = jnp.max(l, axis=1, keepdims=True)
    ex = jnp.exp(l - m)
    att = ex / jnp.sum(ex, axis=1, keepdims=True)
    rc = jnp.sum(att[:, 0:1] * y, axis=0, keepdims=True)
    ro = jnp.sum(att[:, 1:2] * y, axis=0, keepdims=True)
    o_ref[...] += jnp.concatenate([rc, ro], axis=0)


def _readout(y, Watt, batt2):
    return pl.pallas_call(
        _readout_body,
        grid=(_NB,),
        in_specs=[
            pl.BlockSpec((_ROWS, 512), lambda i: (i, 0)),
            pl.BlockSpec((512, 2), lambda i: (0, 0)),
            pl.BlockSpec((1, 2), lambda i: (0, 0)),
        ],
        out_specs=pl.BlockSpec((2, 512), lambda i: (0, 0)),
        out_shape=jax.ShapeDtypeStruct((2, 512), jnp.float32),
    )(y, Watt, batt2)


def _heads_body(r_ref, wc_ref, bc_ref, wo_ref, bo_ref, wco_ref, bco_ref,
                xc_ref, xo_ref, xco_ref):
    rc = r_ref[0:1] * (1.0 / _N)
    ro = r_ref[1:2] * (1.0 / _N)
    xc_ref[...] = jnp.dot(rc, wc_ref[...],
                          preferred_element_type=jnp.float32) + bc_ref[...]
    xo_ref[...] = jnp.dot(ro, wo_ref[...],
                          preferred_element_type=jnp.float32) + bo_ref[...]
    xco_ref[...] = jnp.dot(rc + ro, wco_ref[...],
                           preferred_element_type=jnp.float32) + bco_ref[...]


def _heads(rsums, Wc, bc2, Wo, bo2, Wco, bco2):
    o = jax.ShapeDtypeStruct((1, 10), jnp.float32)
    return pl.pallas_call(
        _heads_body,
        out_shape=[o, o, o],
    )(rsums, Wc, bc2, Wo, bo2, Wco, bco2)


def _edge_phase(z, ss16, sd16, src, dst, nheads):
    ssrc = ss16[:, :nheads]
    sdst = sd16[:, :nheads]
    el = ssrc[src] + sdst[dst]
    el = jnp.where(el >= 0, el, 0.2 * el)
    w = jnp.exp(el)
    denom = jax.ops.segment_sum(w, dst, num_segments=_N)
    zcat = jnp.concatenate(z, axis=1)
    wexp = jnp.repeat(w, 512 // nheads, axis=1)
    outU = jax.ops.segment_sum(zcat[src] * wexp, dst, num_segments=_N)
    u = [outU[:, 0:128], outU[:, 128:256], outU[:, 256:384], outU[:, 384:512]]
    d0 = jnp.pad(denom, ((0, 0), (0, 16 - nheads)))
    d1 = jnp.zeros_like(d0)
    return u, d0, d1


def _blockdiag_attn(a, nheads, hid):
    out = jnp.zeros((nheads * hid, 16), jnp.float32)
    for h in range(nheads):
        out = out.at[h * hid:(h + 1) * hid, h].set(a[h])
    return out


def kernel(h, edge_index, e, W_emb, b_emb, W0, a0, g0, bta0, W1, a1, g1,
           bta1, Wf, af, gf, btf, Watt, batt, Wc, bc, Wo, bo, Wco, bco):
    src = edge_index[0]
    dst = edge_index[1]

    layers = []
    for (W, a, g, bta) in ((W0, a0, g0, bta0), (W1, a1, g1, bta1),
                           (Wf, af, gf, btf)):
        H, d, hid = W.shape
        Wcat = jnp.transpose(W, (1, 0, 2)).reshape(d, H * hid)
        Asrc = _blockdiag_attn(a[:, :hid], H, hid)
        Adst = _blockdiag_attn(a[:, hid:], H, hid)
        Wsd = jnp.concatenate([Wcat @ Asrc, Wcat @ Adst], axis=1)
        layers.append((H, Wcat, Wsd, g.reshape(1, -1), bta.reshape(1, -1)))

    x = _emb_matmul(h, W_emb, b_emb.reshape(1, -1))

    H0, Wcat0, Wsd0, g2, b2 = layers[0]
    z, ss16, sd16 = None, None, None
    outs = _proj(x, Wcat0, Wsd0)
    z, ss16, sd16 = outs[0:4], outs[4], outs[5]

    res = x
    for li in range(3):
        H, Wcat, Wsd, g2, b2 = layers[li]
        u, d0, d1 = _edge_phase(z, ss16, sd16, src, dst, H)
        sums = _stats(u, d0, d1, H)
        do_proj = li < 2
        if do_proj:
            Hn, Wcatn, Wsdn, _, _ = layers[li + 1]
            outs = _apply_proj(u, d0, d1, sums, g2, b2, res, Wcatn, Wsdn,
                               H, True)
            y = outs[0]
            z, ss16, sd16 = outs[1:5], outs[5], outs[6]
        else:
            outs = _apply_proj(u, d0, d1, sums, g2, b2, res, Wcat, Wsd,
                               H, False)
            y = outs[0]
        res = y

    rsums = _readout(y, Watt, batt.reshape(1, -1))
    xc, xo, xco = _heads(rsums, Wc, bc.reshape(1, -1), Wo, bo.reshape(1, -1),
                         Wco, bco.reshape(1, -1))
    return (xc, xo, xco)


# speedup vs baseline: 8.6186x; 1.8960x over previous
import functools

import jax
import jax.numpy as jnp
from jax import lax
from jax.experimental import pallas as pl
from jax.experimental.pallas import tpu as pltpu
from jax.experimental.pallas import tpu_sc as plsc

_N = 10000
_E = 160000
_ROWS = 1000
_NB = _N // _ROWS
_RPT = 624


def _mm_bias_body(x_ref, w_ref, b_ref, o_ref):
    o_ref[...] = jnp.dot(x_ref[...], w_ref[...],
                         preferred_element_type=jnp.float32) + b_ref[...]


def _emb_matmul(h, W, b2):
    k = h.shape[1]
    m = W.shape[1]
    return pl.pallas_call(
        _mm_bias_body,
        grid=(_NB,),
        in_specs=[
            pl.BlockSpec((_ROWS, k), lambda i: (i, 0)),
            pl.BlockSpec((k, m), lambda i: (0, 0)),
            pl.BlockSpec((1, m), lambda i: (0, 0)),
        ],
        out_specs=pl.BlockSpec((_ROWS, m), lambda i: (i, 0)),
        out_shape=jax.ShapeDtypeStruct((_N, m), jnp.float32),
    )(h, W, b2)


def _proj_body(x_ref, wcat_ref, wsd_ref, z_ref, ss_ref, sd_ref):
    x = x_ref[...]
    z_ref[...] = jnp.dot(x, wcat_ref[0], preferred_element_type=jnp.float32)
    s = jnp.dot(x, wsd_ref[...], preferred_element_type=jnp.float32)
    ss_ref[...] = s[:, 0:16]
    sd_ref[...] = s[:, 16:32]


def _proj(x, Wcat8, Wsd):
    outs = [
        jax.ShapeDtypeStruct((8 * _N, 64), jnp.float32),
        jax.ShapeDtypeStruct((_N, 16), jnp.float32),
        jax.ShapeDtypeStruct((_N, 16), jnp.float32),
    ]
    out_specs = [
        pl.BlockSpec((_ROWS, 64), lambda i, s: (s * _NB + i, 0)),
        pl.BlockSpec((_ROWS, 16), lambda i, s: (i, 0)),
        pl.BlockSpec((_ROWS, 16), lambda i, s: (i, 0)),
    ]
    return pl.pallas_call(
        _proj_body,
        grid=(_NB, 8),
        in_specs=[
            pl.BlockSpec((_ROWS, 512), lambda i, s: (i, 0)),
            pl.BlockSpec((1, 512, 64), lambda i, s: (s, 0, 0)),
            pl.BlockSpec((512, 32), lambda i, s: (0, 0)),
        ],
        out_specs=out_specs,
        out_shape=outs,
    )(x, Wcat8, Wsd)


def _t_block(u_refs, d0_ref, d1_ref, nheads):
    d = d0_ref[...] + d1_ref[...]
    cols = []
    for j, u in enumerate(u_refs):
        h = j if nheads == 8 else 0
        cols.append(u[...] / (d[:, h:h + 1] + 1e-9))
    return jnp.concatenate(cols, axis=1)


def _stats_body(u0, u1, u2, u3, u4, u5, u6, u7, d0_ref, d1_ref, o_ref, *,
                nheads):
    i = pl.program_id(0)

    @pl.when(i == 0)
    def _():
        o_ref[...] = jnp.zeros_like(o_ref)

    t = _t_block((u0, u1, u2, u3, u4, u5, u6, u7), d0_ref, d1_ref, nheads)
    s = jnp.sum(t, axis=0, keepdims=True)
    ss = jnp.sum(t * t, axis=0, keepdims=True)
    o_ref[...] += jnp.concatenate([s, ss], axis=0)


def _stats(uf, den, nheads):
    u_specs = [
        pl.BlockSpec((_ROWS, 64), lambda i, k=k: (k * _NB + i, 0))
        for k in range(8)
    ]
    d_specs = [
        pl.BlockSpec((_ROWS, 16), lambda i: (i, 0)),
        pl.BlockSpec((_ROWS, 16), lambda i: (_NB + i, 0)),
    ]
    return pl.pallas_call(
        functools.partial(_stats_body, nheads=nheads),
        grid=(_NB,),
        in_specs=u_specs + d_specs,
        out_specs=pl.BlockSpec((2, 512), lambda i: (0, 0)),
        out_shape=jax.ShapeDtypeStruct((2, 512), jnp.float32),
    )(uf, uf, uf, uf, uf, uf, uf, uf, den, den)


def _y_block(u_refs, d0_ref, d1_ref, sums_ref, g_ref, b_ref, res_ref, nheads):
    t = _t_block(u_refs, d0_ref, d1_ref, nheads)
    sums = sums_ref[...]
    mean = sums[0:1] * (1.0 / _N)
    var = sums[1:2] * (1.0 / _N) - mean * mean
    rstd = lax.rsqrt(var + 1e-5)
    yv = (t - mean) * rstd * g_ref[...] + b_ref[...]
    yv = jnp.where(yv > 0, yv, jnp.exp(jnp.minimum(yv, 0.0)) - 1.0)
    return yv + res_ref[...]


def _apply_proj_body(u0, u1, u2, u3, u4, u5, u6, u7, d0_ref, d1_ref,
                     sums_ref, g_ref, b_ref, res_ref, wcat_ref, wsd_ref,
                     y_ref, z_ref, ss_ref, sd_ref, y_scr, *, nheads):
    s = pl.program_id(1)

    @pl.when(s == 0)
    def _():
        yv = _y_block((u0, u1, u2, u3, u4, u5, u6, u7), d0_ref, d1_ref,
                      sums_ref, g_ref, b_ref, res_ref, nheads)
        y_scr[...] = yv
        y_ref[...] = yv
        sv = jnp.dot(yv, wsd_ref[...], preferred_element_type=jnp.float32)
        ss_ref[...] = sv[:, 0:16]
        sd_ref[...] = sv[:, 16:32]

    z_ref[...] = jnp.dot(y_scr[...], wcat_ref[0],
                         preferred_element_type=jnp.float32)


def _apply_proj(uf, den, sums, g2, b2, res, Wcat8, Wsd, nheads):
    u_specs = [
        pl.BlockSpec((_ROWS, 64), lambda i, s, k=k: (k * _NB + i, 0))
        for k in range(8)
    ]
    other_specs = [
        pl.BlockSpec((_ROWS, 16), lambda i, s: (i, 0)),
        pl.BlockSpec((_ROWS, 16), lambda i, s: (_NB + i, 0)),
        pl.BlockSpec((2, 512), lambda i, s: (0, 0)),
        pl.BlockSpec((1, 512), lambda i, s: (0, 0)),
        pl.BlockSpec((1, 512), lambda i, s: (0, 0)),
        pl.BlockSpec((_ROWS, 512), lambda i, s: (i, 0)),
        pl.BlockSpec((1, 512, 64), lambda i, s: (s, 0, 0)),
        pl.BlockSpec((512, 32), lambda i, s: (0, 0)),
    ]
    outs = [
        jax.ShapeDtypeStruct((_N, 512), jnp.float32),
        jax.ShapeDtypeStruct((8 * _N, 64), jnp.float32),
        jax.ShapeDtypeStruct((_N, 16), jnp.float32),
        jax.ShapeDtypeStruct((_N, 16), jnp.float32),
    ]
    out_specs = [
        pl.BlockSpec((_ROWS, 512), lambda i, s: (i, 0)),
        pl.BlockSpec((_ROWS, 64), lambda i, s: (s * _NB + i, 0)),
        pl.BlockSpec((_ROWS, 16), lambda i, s: (i, 0)),
        pl.BlockSpec((_ROWS, 16), lambda i, s: (i, 0)),
    ]
    return pl.pallas_call(
        functools.partial(_apply_proj_body, nheads=nheads),
        grid=(_NB, 8),
        in_specs=u_specs + other_specs,
        out_specs=out_specs,
        out_shape=outs,
        scratch_shapes=[pltpu.VMEM((_ROWS, 512), jnp.float32)],
    )(uf, uf, uf, uf, uf, uf, uf, uf, den, den, sums, g2, b2, res, Wcat8, Wsd)


def _apply_final_body(u0, u1, u2, u3, u4, u5, u6, u7, d0_ref, d1_ref,
                      sums_ref, g_ref, b_ref, res_ref, y_ref, *, nheads):
    y_ref[...] = _y_block((u0, u1, u2, u3, u4, u5, u6, u7), d0_ref, d1_ref,
                          sums_ref, g_ref, b_ref, res_ref, nheads)


def _apply_final(uf, den, sums, g2, b2, res, nheads):
    u_specs = [
        pl.BlockSpec((_ROWS, 64), lambda i, k=k: (k * _NB + i, 0))
        for k in range(8)
    ]
    other_specs = [
        pl.BlockSpec((_ROWS, 16), lambda i: (i, 0)),
        pl.BlockSpec((_ROWS, 16), lambda i: (_NB + i, 0)),
        pl.BlockSpec((2, 512), lambda i: (0, 0)),
        pl.BlockSpec((1, 512), lambda i: (0, 0)),
        pl.BlockSpec((1, 512), lambda i: (0, 0)),
        pl.BlockSpec((_ROWS, 512), lambda i: (i, 0)),
    ]
    return pl.pallas_call(
        functools.partial(_apply_final_body, nheads=nheads),
        grid=(_NB,),
        in_specs=u_specs + other_specs,
        out_specs=pl.BlockSpec((_ROWS, 512), lambda i: (i, 0)),
        out_shape=jax.ShapeDtypeStruct((_N, 512), jnp.float32),
    )(uf, uf, uf, uf, uf, uf, uf, uf, den, den, sums, g2, b2, res)


def _readout_body(y_ref, watt_ref, batt_ref, o_ref):
    i = pl.program_id(0)

    @pl.when(i == 0)
    def _():
        o_ref[...] = jnp.zeros_like(o_ref)

    y = y_ref[...]
    l = jnp.dot(y, watt_ref[...], preferred_element_type=jnp.float32) \
        + batt_ref[...]
    m = jnp.max(l, axis=1, keepdims=True)
    ex = jnp.exp(l - m)
    att = ex / jnp.sum(ex, axis=1, keepdims=True)
    rc = jnp.sum(att[:, 0:1] * y, axis=0, keepdims=True)
    ro = jnp.sum(att[:, 1:2] * y, axis=0, keepdims=True)
    o_ref[...] += jnp.concatenate([rc, ro], axis=0)


def _readout(y, Watt, batt2):
    return pl.pallas_call(
        _readout_body,
        grid=(_NB,),
        in_specs=[
            pl.BlockSpec((_ROWS, 512), lambda i: (i, 0)),
            pl.BlockSpec((512, 2), lambda i: (0, 0)),
            pl.BlockSpec((1, 2), lambda i: (0, 0)),
        ],
        out_specs=pl.BlockSpec((2, 512), lambda i: (0, 0)),
        out_shape=jax.ShapeDtypeStruct((2, 512), jnp.float32),
    )(y, Watt, batt2)


def _heads_body(r_ref, wc_ref, bc_ref, wo_ref, bo_ref, wco_ref, bco_ref,
                xc_ref, xo_ref, xco_ref):
    rc = r_ref[0:1] * (1.0 / _N)
    ro = r_ref[1:2] * (1.0 / _N)
    xc_ref[...] = jnp.dot(rc, wc_ref[...],
                          preferred_element_type=jnp.float32) + bc_ref[...]
    xo_ref[...] = jnp.dot(ro, wo_ref[...],
                          preferred_element_type=jnp.float32) + bo_ref[...]
    xco_ref[...] = jnp.dot(rc + ro, wco_ref[...],
                           preferred_element_type=jnp.float32) + bco_ref[...]


def _heads(rsums, Wc, bc2, Wo, bo2, Wco, bco2):
    o = jax.ShapeDtypeStruct((1, 10), jnp.float32)
    return pl.pallas_call(
        _heads_body,
        out_shape=[o, o, o],
    )(rsums, Wc, bc2, Wo, bo2, Wco, bco2)


def _phase1_body(src_ref, dst_ref, ssrc_ref, sdst_ref, wexp_ref, den_ref,
                 idx_s, idx_d, rows_a, rows_b, wbuf, zbuf, den_sh, *, nheads):
    c = lax.axis_index("c")
    s = lax.axis_index("s")
    lane = lax.iota(jnp.int32, 16)

    def zfill(i, carry):
        zbuf[i] = jnp.zeros((16,), jnp.float32)
        return carry

    lax.fori_loop(0, 208, zfill, 0)
    for j in range(3):
        pltpu.sync_copy(zbuf, den_sh.at[pl.ds(s * _RPT + j * 208, 208)])

    @pl.when(s == 0)
    def _():
        pltpu.sync_copy(zbuf.at[pl.ds(0, 16)], den_sh.at[pl.ds(16 * _RPT, 16)])

    plsc.subcore_barrier()

    trips = 39 + jnp.where(s == 0, 1, 0)

    def batch(b, carry):
        g = b * 16 + s
        base = c * (_E // 2) + g * 128
        pltpu.sync_copy(src_ref.at[pl.ds(base, 128)], idx_s)
        pltpu.sync_copy(dst_ref.at[pl.ds(base, 128)], idx_d)
        pltpu.sync_copy(ssrc_ref.at[idx_s], rows_a)
        pltpu.sync_copy(sdst_ref.at[idx_d], rows_b)

        def edge(k, ec):
            t = rows_a[k] + rows_b[k]
            l = jnp.where(t >= 0, t, 0.2 * t)
            wbuf[k] = jnp.where(lane < nheads, jnp.exp(l), 0.0)
            return ec

        lax.fori_loop(0, 128, edge, 0)
        pltpu.sync_copy(wbuf, wexp_ref.at[pl.ds(base, 128)])
        pltpu.sync_copy(wbuf, den_sh.at[idx_d], add=True)
        return carry

    lax.fori_loop(0, trips, batch, 0)
    plsc.subcore_barrier()
    pltpu.sync_copy(den_sh.at[pl.ds(s * _RPT, _RPT)],
                    den_ref.at[pl.ds(c * _N + s * _RPT, _RPT)])

    @pl.when(s == 0)
    def _():
        pltpu.sync_copy(den_sh.at[pl.ds(16 * _RPT, 16)],
                        den_ref.at[pl.ds(c * _N + 16 * _RPT, 16)])


def _sc_phase1(src, dst, ss16, sd16, nheads):
    body = functools.partial(_phase1_body, nheads=nheads)
    f = pl.kernel(
        body,
        out_type=[
            jax.ShapeDtypeStruct((_E, 16), jnp.float32),
            jax.ShapeDtypeStruct((2 * _N, 16), jnp.float32),
        ],
        mesh=plsc.VectorSubcoreMesh(core_axis_name="c", subcore_axis_name="s"),
        compiler_params=pltpu.CompilerParams(use_tc_tiling_on_sc=False, needs_layout_passes=False),
        scratch_types=[
            pltpu.VMEM((128,), jnp.int32),
            pltpu.VMEM((128,), jnp.int32),
            pltpu.VMEM((128, 16), jnp.float32),
            pltpu.VMEM((128, 16), jnp.float32),
            pltpu.VMEM((128, 16), jnp.float32),
            pltpu.VMEM((208, 16), jnp.float32),
            pltpu.VMEM_SHARED((_N, 16), jnp.float32),
        ],
    )
    return f(src, dst, ss16, sd16)


def _lane_splat(v, h, lane):
    return jnp.sum(jnp.where(lane == h, v, 0.0))


def _phase2_body(src_ref, dst_ref, wexp_ref, zf_ref, u_ref,
                 idx_s, idx_d, wrows, zrows, obuf, zbuf, acc_sh, *, nheads):
    c = lax.axis_index("c")
    s = lax.axis_index("s")
    lane = lax.iota(jnp.int32, 16)

    def zfill(i, carry):
        for q in range(4):
            zbuf[i, pl.ds(q * 16, 16)] = jnp.zeros((16,), jnp.float32)
        return carry

    trips = 39 + jnp.where(s == 0, 1, 0)

    for sl in range(4):
        slice_id = c * 4 + sl
        lax.fori_loop(0, 208, zfill, 0)
        for j in range(3):
            pltpu.sync_copy(zbuf, acc_sh.at[pl.ds(s * _RPT + j * 208, 208)])

        @pl.when(s == 0)
        def _():
            pltpu.sync_copy(zbuf.at[pl.ds(0, 16)],
                            acc_sh.at[pl.ds(16 * _RPT, 16)])

        plsc.subcore_barrier()

        def batch(b, carry):
            g = b * 16 + s
            base = g * 256
            for j in range(2):
                pltpu.sync_copy(src_ref.at[pl.ds(base + j * 128, 128)],
                                idx_s.at[j])
                pltpu.sync_copy(dst_ref.at[pl.ds(base + j * 128, 128)],
                                idx_d.at[j])
            pltpu.sync_copy(wexp_ref.at[pl.ds(base, 256)], wrows)
            off = slice_id * _N
            for j in range(2):
                for q in range(8):
                    idx_s[j, pl.ds(q * 16, 16)] = \
                        idx_s[j, pl.ds(q * 16, 16)] + off
            for j in range(2):
                pltpu.sync_copy(zf_ref.at[idx_s.at[j]],
                                zrows.at[pl.ds(j * 128, 128)])

            def edge(k, ec):
                wrow = wrows[k]
                if nheads == 8:
                    sc = _lane_splat(wrow, slice_id, lane)
                else:
                    sc = _lane_splat(wrow, 0, lane)
                for q in range(4):
                    obuf[k, pl.ds(q * 16, 16)] = \
                        zrows[k, pl.ds(q * 16, 16)] * sc
                return ec

            lax.fori_loop(0, 256, edge, 0)
            for j in range(2):
                pltpu.sync_copy(obuf.at[pl.ds(j * 128, 128)],
                                acc_sh.at[idx_d.at[j]], add=True)
            return carry

        lax.fori_loop(0, trips, batch, 0)
        plsc.subcore_barrier()
        pltpu.sync_copy(acc_sh.at[pl.ds(s * _RPT, _RPT)],
                        u_ref.at[pl.ds(slice_id * _N + s * _RPT, _RPT)])

        @pl.when(s == 0)
        def _():
            pltpu.sync_copy(acc_sh.at[pl.ds(16 * _RPT, 16)],
                            u_ref.at[pl.ds(slice_id * _N + 16 * _RPT, 16)])

        if sl < 3:
            plsc.subcore_barrier()


def _sc_phase2(src, dst, wexp, zf, nheads):
    body = functools.partial(_phase2_body, nheads=nheads)
    f = pl.kernel(
        body,
        out_type=jax.ShapeDtypeStruct((8 * _N, 64), jnp.float32),
        mesh=plsc.VectorSubcoreMesh(core_axis_name="c", subcore_axis_name="s"),
        compiler_params=pltpu.CompilerParams(use_tc_tiling_on_sc=False, needs_layout_passes=False),
        scratch_types=[
            pltpu.VMEM((2, 128), jnp.int32),
            pltpu.VMEM((2, 128), jnp.int32),
            pltpu.VMEM((256, 16), jnp.float32),
            pltpu.VMEM((256, 64), jnp.float32),
            pltpu.VMEM((256, 64), jnp.float32),
            pltpu.VMEM((208, 64), jnp.float32),
            pltpu.VMEM_SHARED((_N, 64), jnp.float32),
        ],
    )
    return f(src, dst, wexp, zf)


def _blockdiag_attn(a, nheads, hid):
    out = jnp.zeros((nheads * hid, 16), jnp.float32)
    for h in range(nheads):
        out = out.at[h * hid:(h + 1) * hid, h].set(a[h])
    return out


def kernel(h, edge_index, e, W_emb, b_emb, W0, a0, g0, bta0, W1, a1, g1,
           bta1, Wf, af, gf, btf, Watt, batt, Wc, bc, Wo, bo, Wco, bco):
    src = edge_index[0]
    dst = edge_index[1]

    layers = []
    for (W, a, g, bta) in ((W0, a0, g0, bta0), (W1, a1, g1, bta1),
                           (Wf, af, gf, btf)):
        H, d, hid = W.shape
        Wcat = jnp.transpose(W, (1, 0, 2)).reshape(d, H * hid)
        Asrc = _blockdiag_attn(a[:, :hid], H, hid)
        Adst = _blockdiag_attn(a[:, hid:], H, hid)
        Wsd = jnp.concatenate([Wcat @ Asrc, Wcat @ Adst], axis=1)
        Wcat8 = jnp.transpose(Wcat.reshape(d, 8, 64), (1, 0, 2))
        layers.append((H, Wcat8, Wsd, g.reshape(1, -1), bta.reshape(1, -1)))

    x = _emb_matmul(h, W_emb, b_emb.reshape(1, -1))

    H0, Wcat80, Wsd0, _, _ = layers[0]
    zf, ss16, sd16 = _proj(x, Wcat80, Wsd0)

    res = x
    y = None
    for li in range(3):
        H, _, Wsd, g2, b2 = layers[li]
        wexp, den = _sc_phase1(src, dst, ss16, sd16, H)
        uf = _sc_phase2(src, dst, wexp, zf, H)
        sums = _stats(uf, den, H)
        if li < 2:
            Hn, Wcat8n, Wsdn, _, _ = layers[li + 1]
            y, zf, ss16, sd16 = _apply_proj(uf, den, sums, g2, b2, res,
                                            Wcat8n, Wsdn, H)
        else:
            y = _apply_final(uf, den, sums, g2, b2, res, H)
        res = y

    rsums = _readout(y, Watt, batt.reshape(1, -1))
    xc, xo, xco = _heads(rsums, Wc, bc.reshape(1, -1), Wo, bo.reshape(1, -1),
                         Wco, bco.reshape(1, -1))
    return (xc, xo, xco)


# speedup vs baseline: 16.9862x; 1.9709x over previous
import functools

import jax
import jax.numpy as jnp
from jax import lax
from jax.experimental import pallas as pl
from jax.experimental.pallas import tpu as pltpu
from jax.experimental.pallas import tpu_sc as plsc

_N = 10000
_E = 160000
_ROWS = 1000
_NB = _N // _ROWS
_RPT = 624


def _mm_bias_body(x_ref, w_ref, b_ref, o_ref):
    o_ref[...] = jnp.dot(x_ref[...], w_ref[...],
                         preferred_element_type=jnp.float32) + b_ref[...]


def _emb_matmul(h, W, b2):
    k = h.shape[1]
    m = W.shape[1]
    return pl.pallas_call(
        _mm_bias_body,
        grid=(_NB,),
        in_specs=[
            pl.BlockSpec((_ROWS, k), lambda i: (i, 0)),
            pl.BlockSpec((k, m), lambda i: (0, 0)),
            pl.BlockSpec((1, m), lambda i: (0, 0)),
        ],
        out_specs=pl.BlockSpec((_ROWS, m), lambda i: (i, 0)),
        out_shape=jax.ShapeDtypeStruct((_N, m), jnp.float32),
    )(h, W, b2)


def _proj_body(x_ref, wcat_ref, wsd_ref, z_ref, ss_ref, sd_ref):
    x = x_ref[...]
    z_ref[...] = jnp.dot(x, wcat_ref[0], preferred_element_type=jnp.float32)
    s = jnp.dot(x, wsd_ref[...], preferred_element_type=jnp.float32)
    ss_ref[...] = s[:, 0:16]
    sd_ref[...] = s[:, 16:32]


def _proj(x, Wcat8, Wsd):
    outs = [
        jax.ShapeDtypeStruct((8 * _N, 64), jnp.float32),
        jax.ShapeDtypeStruct((_N, 16), jnp.float32),
        jax.ShapeDtypeStruct((_N, 16), jnp.float32),
    ]
    out_specs = [
        pl.BlockSpec((_ROWS, 64), lambda i, s: (s * _NB + i, 0)),
        pl.BlockSpec((_ROWS, 16), lambda i, s: (i, 0)),
        pl.BlockSpec((_ROWS, 16), lambda i, s: (i, 0)),
    ]
    return pl.pallas_call(
        _proj_body,
        grid=(_NB, 8),
        in_specs=[
            pl.BlockSpec((_ROWS, 512), lambda i, s: (i, 0)),
            pl.BlockSpec((1, 512, 64), lambda i, s: (s, 0, 0)),
            pl.BlockSpec((512, 32), lambda i, s: (0, 0)),
        ],
        out_specs=out_specs,
        out_shape=outs,
    )(x, Wcat8, Wsd)


def _t_block(u_refs, d0_ref, d1_ref, nheads):
    d = d0_ref[...] + d1_ref[...]
    cols = []
    for j, u in enumerate(u_refs):
        h = j if nheads == 8 else 0
        cols.append(u[...] / (d[:, h:h + 1] + 1e-9))
    return jnp.concatenate(cols, axis=1)


def _stats_body(u0, u1, u2, u3, u4, u5, u6, u7, d0_ref, d1_ref, o_ref, *,
                nheads):
    i = pl.program_id(0)

    @pl.when(i == 0)
    def _():
        o_ref[...] = jnp.zeros_like(o_ref)

    t = _t_block((u0, u1, u2, u3, u4, u5, u6, u7), d0_ref, d1_ref, nheads)
    s = jnp.sum(t, axis=0, keepdims=True)
    ss = jnp.sum(t * t, axis=0, keepdims=True)
    o_ref[...] += jnp.concatenate([s, ss], axis=0)


def _stats(uf, den, nheads):
    u_specs = [
        pl.BlockSpec((_ROWS, 64), lambda i, k=k: (k * _NB + i, 0))
        for k in range(8)
    ]
    d_specs = [
        pl.BlockSpec((_ROWS, 16), lambda i: (i, 0)),
        pl.BlockSpec((_ROWS, 16), lambda i: (_NB + i, 0)),
    ]
    return pl.pallas_call(
        functools.partial(_stats_body, nheads=nheads),
        grid=(_NB,),
        in_specs=u_specs + d_specs,
        out_specs=pl.BlockSpec((2, 512), lambda i: (0, 0)),
        out_shape=jax.ShapeDtypeStruct((2, 512), jnp.float32),
    )(uf, uf, uf, uf, uf, uf, uf, uf, den, den)


def _y_block(u_refs, d0_ref, d1_ref, sums_ref, g_ref, b_ref, res_ref, nheads):
    t = _t_block(u_refs, d0_ref, d1_ref, nheads)
    sums = sums_ref[...]
    mean = sums[0:1] * (1.0 / _N)
    var = sums[1:2] * (1.0 / _N) - mean * mean
    rstd = lax.rsqrt(var + 1e-5)
    yv = (t - mean) * rstd * g_ref[...] + b_ref[...]
    yv = jnp.where(yv > 0, yv, jnp.exp(jnp.minimum(yv, 0.0)) - 1.0)
    return yv + res_ref[...]


def _apply_proj_body(u0, u1, u2, u3, u4, u5, u6, u7, d0_ref, d1_ref,
                     sums_ref, g_ref, b_ref, res_ref, wcat_ref, wsd_ref,
                     y_ref, z_ref, ss_ref, sd_ref, y_scr, *, nheads):
    s = pl.program_id(1)

    @pl.when(s == 0)
    def _():
        yv = _y_block((u0, u1, u2, u3, u4, u5, u6, u7), d0_ref, d1_ref,
                      sums_ref, g_ref, b_ref, res_ref, nheads)
        y_scr[...] = yv
        y_ref[...] = yv
        sv = jnp.dot(yv, wsd_ref[...], preferred_element_type=jnp.float32)
        ss_ref[...] = sv[:, 0:16]
        sd_ref[...] = sv[:, 16:32]

    z_ref[...] = jnp.dot(y_scr[...], wcat_ref[0],
                         preferred_element_type=jnp.float32)


def _apply_proj(uf, den, sums, g2, b2, res, Wcat8, Wsd, nheads):
    u_specs = [
        pl.BlockSpec((_ROWS, 64), lambda i, s, k=k: (k * _NB + i, 0))
        for k in range(8)
    ]
    other_specs = [
        pl.BlockSpec((_ROWS, 16), lambda i, s: (i, 0)),
        pl.BlockSpec((_ROWS, 16), lambda i, s: (_NB + i, 0)),
        pl.BlockSpec((2, 512), lambda i, s: (0, 0)),
        pl.BlockSpec((1, 512), lambda i, s: (0, 0)),
        pl.BlockSpec((1, 512), lambda i, s: (0, 0)),
        pl.BlockSpec((_ROWS, 512), lambda i, s: (i, 0)),
        pl.BlockSpec((1, 512, 64), lambda i, s: (s, 0, 0)),
        pl.BlockSpec((512, 32), lambda i, s: (0, 0)),
    ]
    outs = [
        jax.ShapeDtypeStruct((_N, 512), jnp.float32),
        jax.ShapeDtypeStruct((8 * _N, 64), jnp.float32),
        jax.ShapeDtypeStruct((_N, 16), jnp.float32),
        jax.ShapeDtypeStruct((_N, 16), jnp.float32),
    ]
    out_specs = [
        pl.BlockSpec((_ROWS, 512), lambda i, s: (i, 0)),
        pl.BlockSpec((_ROWS, 64), lambda i, s: (s * _NB + i, 0)),
        pl.BlockSpec((_ROWS, 16), lambda i, s: (i, 0)),
        pl.BlockSpec((_ROWS, 16), lambda i, s: (i, 0)),
    ]
    return pl.pallas_call(
        functools.partial(_apply_proj_body, nheads=nheads),
        grid=(_NB, 8),
        in_specs=u_specs + other_specs,
        out_specs=out_specs,
        out_shape=outs,
        scratch_shapes=[pltpu.VMEM((_ROWS, 512), jnp.float32)],
    )(uf, uf, uf, uf, uf, uf, uf, uf, den, den, sums, g2, b2, res, Wcat8, Wsd)


def _apply_final_body(u0, u1, u2, u3, u4, u5, u6, u7, d0_ref, d1_ref,
                      sums_ref, g_ref, b_ref, res_ref, y_ref, *, nheads):
    y_ref[...] = _y_block((u0, u1, u2, u3, u4, u5, u6, u7), d0_ref, d1_ref,
                          sums_ref, g_ref, b_ref, res_ref, nheads)


def _apply_final(uf, den, sums, g2, b2, res, nheads):
    u_specs = [
        pl.BlockSpec((_ROWS, 64), lambda i, k=k: (k * _NB + i, 0))
        for k in range(8)
    ]
    other_specs = [
        pl.BlockSpec((_ROWS, 16), lambda i: (i, 0)),
        pl.BlockSpec((_ROWS, 16), lambda i: (_NB + i, 0)),
        pl.BlockSpec((2, 512), lambda i: (0, 0)),
        pl.BlockSpec((1, 512), lambda i: (0, 0)),
        pl.BlockSpec((1, 512), lambda i: (0, 0)),
        pl.BlockSpec((_ROWS, 512), lambda i: (i, 0)),
    ]
    return pl.pallas_call(
        functools.partial(_apply_final_body, nheads=nheads),
        grid=(_NB,),
        in_specs=u_specs + other_specs,
        out_specs=pl.BlockSpec((_ROWS, 512), lambda i: (i, 0)),
        out_shape=jax.ShapeDtypeStruct((_N, 512), jnp.float32),
    )(uf, uf, uf, uf, uf, uf, uf, uf, den, den, sums, g2, b2, res)


def _readout_body(y_ref, watt_ref, batt_ref, o_ref):
    i = pl.program_id(0)

    @pl.when(i == 0)
    def _():
        o_ref[...] = jnp.zeros_like(o_ref)

    y = y_ref[...]
    l = jnp.dot(y, watt_ref[...], preferred_element_type=jnp.float32) \
        + batt_ref[...]
    m = jnp.max(l, axis=1, keepdims=True)
    ex = jnp.exp(l - m)
    att = ex / jnp.sum(ex, axis=1, keepdims=True)
    rc = jnp.sum(att[:, 0:1] * y, axis=0, keepdims=True)
    ro = jnp.sum(att[:, 1:2] * y, axis=0, keepdims=True)
    o_ref[...] += jnp.concatenate([rc, ro], axis=0)


def _readout(y, Watt, batt2):
    return pl.pallas_call(
        _readout_body,
        grid=(_NB,),
        in_specs=[
            pl.BlockSpec((_ROWS, 512), lambda i: (i, 0)),
            pl.BlockSpec((512, 2), lambda i: (0, 0)),
            pl.BlockSpec((1, 2), lambda i: (0, 0)),
        ],
        out_specs=pl.BlockSpec((2, 512), lambda i: (0, 0)),
        out_shape=jax.ShapeDtypeStruct((2, 512), jnp.float32),
    )(y, Watt, batt2)


def _heads_body(r_ref, wc_ref, bc_ref, wo_ref, bo_ref, wco_ref, bco_ref,
                xc_ref, xo_ref, xco_ref):
    rc = r_ref[0:1] * (1.0 / _N)
    ro = r_ref[1:2] * (1.0 / _N)
    xc_ref[...] = jnp.dot(rc, wc_ref[...],
                          preferred_element_type=jnp.float32) + bc_ref[...]
    xo_ref[...] = jnp.dot(ro, wo_ref[...],
                          preferred_element_type=jnp.float32) + bo_ref[...]
    xco_ref[...] = jnp.dot(rc + ro, wco_ref[...],
                           preferred_element_type=jnp.float32) + bco_ref[...]


def _heads(rsums, Wc, bc2, Wo, bo2, Wco, bco2):
    o = jax.ShapeDtypeStruct((1, 10), jnp.float32)
    return pl.pallas_call(
        _heads_body,
        out_shape=[o, o, o],
    )(rsums, Wc, bc2, Wo, bo2, Wco, bco2)


def _phase1_body(src_ref, dst_ref, ssrc_ref, sdst_ref, wexp_ref, den_ref,
                 idx_s, idx_d, rows_a, rows_b, wbuf, zbuf, den_sh, *, nheads):
    c = lax.axis_index("c")
    s = lax.axis_index("s")
    lane = lax.iota(jnp.int32, 16)

    def zfill(i, carry):
        zbuf[i] = jnp.zeros((16,), jnp.float32)
        return carry

    lax.fori_loop(0, 208, zfill, 0)
    for j in range(3):
        pltpu.sync_copy(zbuf, den_sh.at[pl.ds(s * _RPT + j * 208, 208)])

    @pl.when(s == 0)
    def _():
        pltpu.sync_copy(zbuf.at[pl.ds(0, 16)], den_sh.at[pl.ds(16 * _RPT, 16)])

    plsc.subcore_barrier()

    trips = 39 + jnp.where(s == 0, 1, 0)

    def batch(b, carry):
        g = b * 16 + s
        base = c * (_E // 2) + g * 128
        pltpu.sync_copy(src_ref.at[pl.ds(base, 128)], idx_s)
        pltpu.sync_copy(dst_ref.at[pl.ds(base, 128)], idx_d)
        pltpu.sync_copy(ssrc_ref.at[idx_s], rows_a)
        pltpu.sync_copy(sdst_ref.at[idx_d], rows_b)

        def edge(k, ec):
            t = rows_a[k] + rows_b[k]
            l = jnp.where(t >= 0, t, 0.2 * t)
            wbuf[k] = jnp.where(lane < nheads, jnp.exp(l), 0.0)
            return ec

        lax.fori_loop(0, 128, edge, 0)
        pltpu.sync_copy(wbuf, wexp_ref.at[pl.ds(base, 128)])
        pltpu.sync_copy(wbuf, den_sh.at[idx_d], add=True)
        return carry

    lax.fori_loop(0, trips, batch, 0)
    plsc.subcore_barrier()
    pltpu.sync_copy(den_sh.at[pl.ds(s * _RPT, _RPT)],
                    den_ref.at[pl.ds(c * _N + s * _RPT, _RPT)])

    @pl.when(s == 0)
    def _():
        pltpu.sync_copy(den_sh.at[pl.ds(16 * _RPT, 16)],
                        den_ref.at[pl.ds(c * _N + 16 * _RPT, 16)])


def _sc_phase1(src, dst, ss16, sd16, nheads):
    body = functools.partial(_phase1_body, nheads=nheads)
    f = pl.kernel(
        body,
        out_type=[
            jax.ShapeDtypeStruct((_E, 16), jnp.float32),
            jax.ShapeDtypeStruct((2 * _N, 16), jnp.float32),
        ],
        mesh=plsc.VectorSubcoreMesh(core_axis_name="c", subcore_axis_name="s"),
        compiler_params=pltpu.CompilerParams(use_tc_tiling_on_sc=False, needs_layout_passes=False),
        scratch_types=[
            pltpu.VMEM((128,), jnp.int32),
            pltpu.VMEM((128,), jnp.int32),
            pltpu.VMEM((128, 16), jnp.float32),
            pltpu.VMEM((128, 16), jnp.float32),
            pltpu.VMEM((128, 16), jnp.float32),
            pltpu.VMEM((208, 16), jnp.float32),
            pltpu.VMEM_SHARED((_N, 16), jnp.float32),
        ],
    )
    return f(src, dst, ss16, sd16)


def _lane_splat(v, h):
    idxv = jnp.full((16,), h, jnp.int32)
    return lax.gather(
        v, idxv[:, None],
        lax.GatherDimensionNumbers(offset_dims=(), collapsed_slice_dims=(0,),
                                   start_index_map=(0,)),
        slice_sizes=(1,), mode=lax.GatherScatterMode.PROMISE_IN_BOUNDS)


def _phase2_body(src_ref, dst_ref, wexp_ref, zf_ref, u_ref,
                 isA, idA, wrA, zrA, isB, idB, wrB, zrB,
                 obuf, zbuf, acc_sh, semiA, semgA, semiB, semgB, *, nheads):
    c = lax.axis_index("c")
    s = lax.axis_index("s")

    def zfill(i, carry):
        for q in range(4):
            zbuf[i, pl.ds(q * 16, 16)] = jnp.zeros((16,), jnp.float32)
        return carry

    trips = 39 + jnp.where(s == 0, 1, 0)

    def stage1(b, idx_s, idx_d, wrows, semi):
        base = (b * 16 + s) * 256
        for j in range(2):
            pltpu.async_copy(src_ref.at[pl.ds(base + j * 128, 128)],
                             idx_s.at[j], semi)
            pltpu.async_copy(dst_ref.at[pl.ds(base + j * 128, 128)],
                             idx_d.at[j], semi)
        pltpu.async_copy(wexp_ref.at[pl.ds(base, 256)], wrows, semi)

    def wait_stage1(idx_s, idx_d, wrows, semi):
        for j in range(2):
            pltpu.make_async_copy(src_ref.at[pl.ds(0, 128)],
                                  idx_s.at[j], semi).wait()
            pltpu.make_async_copy(dst_ref.at[pl.ds(0, 128)],
                                  idx_d.at[j], semi).wait()
        pltpu.make_async_copy(wexp_ref.at[pl.ds(0, 256)], wrows, semi).wait()

    def fire_gather(idx_s, zrows, semg, off):
        for j in range(2):
            for q in range(8):
                idx_s[j, pl.ds(q * 16, 16)] = \
                    idx_s[j, pl.ds(q * 16, 16)] + off
        for j in range(2):
            pltpu.async_copy(zf_ref.at[idx_s.at[j]],
                             zrows.at[pl.ds(j * 128, 128)], semg)

    def wait_gather(zrows, semg):
        for j in range(2):
            pltpu.make_async_copy(zf_ref.at[pl.ds(0, 128)],
                                  zrows.at[pl.ds(j * 128, 128)], semg).wait()

    for sl in range(4):
        slice_id = c * 4 + sl
        off = slice_id * _N
        lax.fori_loop(0, 208, zfill, 0)
        for j in range(3):
            pltpu.sync_copy(zbuf, acc_sh.at[pl.ds(s * _RPT + j * 208, 208)])

        @pl.when(s == 0)
        def _():
            pltpu.sync_copy(zbuf.at[pl.ds(0, 16)],
                            acc_sh.at[pl.ds(16 * _RPT, 16)])

        plsc.subcore_barrier()

        stage1(0, isA, idA, wrA, semiA)
        wait_stage1(isA, idA, wrA, semiA)
        fire_gather(isA, zrA, semgA, off)

        def process(b, idx_s, idx_d, wrows, zrows, semi, semg,
                    nidx_s, nidx_d, nwrows, nzrows, nsemi, nsemg):
            @pl.when(b + 1 < trips)
            def _():
                stage1(b + 1, nidx_s, nidx_d, nwrows, nsemi)

            wait_gather(zrows, semg)

            def edge(k, ec):
                wrow = wrows[k]
                if nheads == 8:
                    scv = _lane_splat(wrow, slice_id)
                else:
                    scv = _lane_splat(wrow, 0)
                for q in range(4):
                    obuf[k, pl.ds(q * 16, 16)] = \
                        zrows[k, pl.ds(q * 16, 16)] * scv
                return ec

            lax.fori_loop(0, 256, edge, 0)

            @pl.when(b + 1 < trips)
            def _():
                wait_stage1(nidx_s, nidx_d, nwrows, nsemi)
                fire_gather(nidx_s, nzrows, nsemg, off)

            for j in range(2):
                pltpu.sync_copy(obuf.at[pl.ds(j * 128, 128)],
                                acc_sh.at[idx_d.at[j]], add=True)

        def piter(b, carry):
            @pl.when(b % 2 == 0)
            def _():
                process(b, isA, idA, wrA, zrA, semiA, semgA,
                        isB, idB, wrB, zrB, semiB, semgB)

            @pl.when(b % 2 == 1)
            def _():
                process(b, isB, idB, wrB, zrB, semiB, semgB,
                        isA, idA, wrA, zrA, semiA, semgA)

            return carry

        lax.fori_loop(0, trips, piter, 0)
        plsc.subcore_barrier()
        pltpu.sync_copy(acc_sh.at[pl.ds(s * _RPT, _RPT)],
                        u_ref.at[pl.ds(slice_id * _N + s * _RPT, _RPT)])

        @pl.when(s == 0)
        def _():
            pltpu.sync_copy(acc_sh.at[pl.ds(16 * _RPT, 16)],
                            u_ref.at[pl.ds(slice_id * _N + 16 * _RPT, 16)])

        if sl < 3:
            plsc.subcore_barrier()


def _sc_phase2(src, dst, wexp, zf, nheads):
    body = functools.partial(_phase2_body, nheads=nheads)
    f = pl.kernel(
        body,
        out_type=jax.ShapeDtypeStruct((8 * _N, 64), jnp.float32),
        mesh=plsc.VectorSubcoreMesh(core_axis_name="c", subcore_axis_name="s"),
        compiler_params=pltpu.CompilerParams(use_tc_tiling_on_sc=False, needs_layout_passes=False),
        scratch_types=[
            pltpu.VMEM((2, 128), jnp.int32),
            pltpu.VMEM((2, 128), jnp.int32),
            pltpu.VMEM((256, 16), jnp.float32),
            pltpu.VMEM((256, 64), jnp.float32),
            pltpu.VMEM((2, 128), jnp.int32),
            pltpu.VMEM((2, 128), jnp.int32),
            pltpu.VMEM((256, 16), jnp.float32),
            pltpu.VMEM((256, 64), jnp.float32),
            pltpu.VMEM((256, 64), jnp.float32),
            pltpu.VMEM((208, 64), jnp.float32),
            pltpu.VMEM_SHARED((_N, 64), jnp.float32),
            pltpu.SemaphoreType.DMA,
            pltpu.SemaphoreType.DMA,
            pltpu.SemaphoreType.DMA,
            pltpu.SemaphoreType.DMA,
        ],
    )
    return f(src, dst, wexp, zf)


def _blockdiag_attn(a, nheads, hid):
    out = jnp.zeros((nheads * hid, 16), jnp.float32)
    for h in range(nheads):
        out = out.at[h * hid:(h + 1) * hid, h].set(a[h])
    return out


def kernel(h, edge_index, e, W_emb, b_emb, W0, a0, g0, bta0, W1, a1, g1,
           bta1, Wf, af, gf, btf, Watt, batt, Wc, bc, Wo, bo, Wco, bco):
    src = edge_index[0]
    dst = edge_index[1]

    layers = []
    for (W, a, g, bta) in ((W0, a0, g0, bta0), (W1, a1, g1, bta1),
                           (Wf, af, gf, btf)):
        H, d, hid = W.shape
        Wcat = jnp.transpose(W, (1, 0, 2)).reshape(d, H * hid)
        Asrc = _blockdiag_attn(a[:, :hid], H, hid)
        Adst = _blockdiag_attn(a[:, hid:], H, hid)
        Wsd = jnp.concatenate([Wcat @ Asrc, Wcat @ Adst], axis=1)
        Wcat8 = jnp.transpose(Wcat.reshape(d, 8, 64), (1, 0, 2))
        layers.append((H, Wcat8, Wsd, g.reshape(1, -1), bta.reshape(1, -1)))

    x = _emb_matmul(h, W_emb, b_emb.reshape(1, -1))

    H0, Wcat80, Wsd0, _, _ = layers[0]
    zf, ss16, sd16 = _proj(x, Wcat80, Wsd0)

    res = x
    y = None
    for li in range(3):
        H, _, Wsd, g2, b2 = layers[li]
        wexp, den = _sc_phase1(src, dst, ss16, sd16, H)
        uf = _sc_phase2(src, dst, wexp, zf, H)
        sums = _stats(uf, den, H)
        if li < 2:
            Hn, Wcat8n, Wsdn, _, _ = layers[li + 1]
            y, zf, ss16, sd16 = _apply_proj(uf, den, sums, g2, b2, res,
                                            Wcat8n, Wsdn, H)
        else:
            y = _apply_final(uf, den, sums, g2, b2, res, H)
        res = y

    rsums = _readout(y, Watt, batt.reshape(1, -1))
    xc, xo, xco = _heads(rsums, Wc, bc.reshape(1, -1), Wo, bo.reshape(1, -1),
                         Wco, bco.reshape(1, -1))
    return (xc, xo, xco)


# speedup vs baseline: 18.3585x; 1.0808x over previous
import functools

import jax
import jax.numpy as jnp
from jax import lax
from jax.experimental import pallas as pl
from jax.experimental.pallas import tpu as pltpu
from jax.experimental.pallas import tpu_sc as plsc

_N = 10000
_E = 160000
_ROWS = 1000
_NB = _N // _ROWS
_RPT = 624


def _mm_bias_body(x_ref, w_ref, b_ref, o_ref):
    o_ref[...] = jnp.dot(x_ref[...], w_ref[...],
                         preferred_element_type=jnp.float32) + b_ref[...]


def _emb_matmul(h, W, b2):
    k = h.shape[1]
    m = W.shape[1]
    return pl.pallas_call(
        _mm_bias_body,
        grid=(_NB,),
        in_specs=[
            pl.BlockSpec((_ROWS, k), lambda i: (i, 0)),
            pl.BlockSpec((k, m), lambda i: (0, 0)),
            pl.BlockSpec((1, m), lambda i: (0, 0)),
        ],
        out_specs=pl.BlockSpec((_ROWS, m), lambda i: (i, 0)),
        out_shape=jax.ShapeDtypeStruct((_N, m), jnp.float32),
    )(h, W, b2)


def _proj_body(x_ref, wcat_ref, wsd_ref, z_ref, ss_ref, sd_ref):
    x = x_ref[...]
    z_ref[...] = jnp.dot(x, wcat_ref[0], preferred_element_type=jnp.float32)
    s = jnp.dot(x, wsd_ref[...], preferred_element_type=jnp.float32)
    ss_ref[...] = s[:, 0:16]
    sd_ref[...] = s[:, 16:32]


def _proj(x, Wcat8, Wsd):
    outs = [
        jax.ShapeDtypeStruct((8 * _N, 64), jnp.float32),
        jax.ShapeDtypeStruct((_N, 16), jnp.float32),
        jax.ShapeDtypeStruct((_N, 16), jnp.float32),
    ]
    out_specs = [
        pl.BlockSpec((_ROWS, 64), lambda i, s: (s * _NB + i, 0)),
        pl.BlockSpec((_ROWS, 16), lambda i, s: (i, 0)),
        pl.BlockSpec((_ROWS, 16), lambda i, s: (i, 0)),
    ]
    return pl.pallas_call(
        _proj_body,
        grid=(_NB, 8),
        in_specs=[
            pl.BlockSpec((_ROWS, 512), lambda i, s: (i, 0)),
            pl.BlockSpec((1, 512, 64), lambda i, s: (s, 0, 0)),
            pl.BlockSpec((512, 32), lambda i, s: (0, 0)),
        ],
        out_specs=out_specs,
        out_shape=outs,
    )(x, Wcat8, Wsd)


def _t_block(u_refs, d0_ref, d1_ref, nheads):
    d = d0_ref[...] + d1_ref[...]
    cols = []
    for j, u in enumerate(u_refs):
        h = j if nheads == 8 else 0
        cols.append(u[...] / (d[:, h:h + 1] + 1e-9))
    return jnp.concatenate(cols, axis=1)


def _stats_body(u0, u1, u2, u3, u4, u5, u6, u7, d0_ref, d1_ref, o_ref, *,
                nheads):
    i = pl.program_id(0)

    @pl.when(i == 0)
    def _():
        o_ref[...] = jnp.zeros_like(o_ref)

    t = _t_block((u0, u1, u2, u3, u4, u5, u6, u7), d0_ref, d1_ref, nheads)
    s = jnp.sum(t, axis=0, keepdims=True)
    ss = jnp.sum(t * t, axis=0, keepdims=True)
    o_ref[...] += jnp.concatenate([s, ss], axis=0)


def _stats(uf, den, nheads):
    u_specs = [
        pl.BlockSpec((_ROWS, 64), lambda i, k=k: (k * _NB + i, 0))
        for k in range(8)
    ]
    d_specs = [
        pl.BlockSpec((_ROWS, 16), lambda i: (i, 0)),
        pl.BlockSpec((_ROWS, 16), lambda i: (_NB + i, 0)),
    ]
    return pl.pallas_call(
        functools.partial(_stats_body, nheads=nheads),
        grid=(_NB,),
        in_specs=u_specs + d_specs,
        out_specs=pl.BlockSpec((2, 512), lambda i: (0, 0)),
        out_shape=jax.ShapeDtypeStruct((2, 512), jnp.float32),
    )(uf, uf, uf, uf, uf, uf, uf, uf, den, den)


def _y_block(u_refs, d0_ref, d1_ref, sums_ref, g_ref, b_ref, res_ref, nheads):
    t = _t_block(u_refs, d0_ref, d1_ref, nheads)
    sums = sums_ref[...]
    mean = sums[0:1] * (1.0 / _N)
    var = sums[1:2] * (1.0 / _N) - mean * mean
    rstd = lax.rsqrt(var + 1e-5)
    yv = (t - mean) * rstd * g_ref[...] + b_ref[...]
    yv = jnp.where(yv > 0, yv, jnp.exp(jnp.minimum(yv, 0.0)) - 1.0)
    return yv + res_ref[...]


def _apply_proj_body(u0, u1, u2, u3, u4, u5, u6, u7, d0_ref, d1_ref,
                     sums_ref, g_ref, b_ref, res_ref, wcat_ref, wsd_ref,
                     y_ref, z_ref, ss_ref, sd_ref, y_scr, *, nheads):
    s = pl.program_id(1)

    @pl.when(s == 0)
    def _():
        yv = _y_block((u0, u1, u2, u3, u4, u5, u6, u7), d0_ref, d1_ref,
                      sums_ref, g_ref, b_ref, res_ref, nheads)
        y_scr[...] = yv
        y_ref[...] = yv
        sv = jnp.dot(yv, wsd_ref[...], preferred_element_type=jnp.float32)
        ss_ref[...] = sv[:, 0:16]
        sd_ref[...] = sv[:, 16:32]

    z_ref[...] = jnp.dot(y_scr[...], wcat_ref[0],
                         preferred_element_type=jnp.float32)


def _apply_proj(uf, den, sums, g2, b2, res, Wcat8, Wsd, nheads):
    u_specs = [
        pl.BlockSpec((_ROWS, 64), lambda i, s, k=k: (k * _NB + i, 0))
        for k in range(8)
    ]
    other_specs = [
        pl.BlockSpec((_ROWS, 16), lambda i, s: (i, 0)),
        pl.BlockSpec((_ROWS, 16), lambda i, s: (_NB + i, 0)),
        pl.BlockSpec((2, 512), lambda i, s: (0, 0)),
        pl.BlockSpec((1, 512), lambda i, s: (0, 0)),
        pl.BlockSpec((1, 512), lambda i, s: (0, 0)),
        pl.BlockSpec((_ROWS, 512), lambda i, s: (i, 0)),
        pl.BlockSpec((1, 512, 64), lambda i, s: (s, 0, 0)),
        pl.BlockSpec((512, 32), lambda i, s: (0, 0)),
    ]
    outs = [
        jax.ShapeDtypeStruct((_N, 512), jnp.float32),
        jax.ShapeDtypeStruct((8 * _N, 64), jnp.float32),
        jax.ShapeDtypeStruct((_N, 16), jnp.float32),
        jax.ShapeDtypeStruct((_N, 16), jnp.float32),
    ]
    out_specs = [
        pl.BlockSpec((_ROWS, 512), lambda i, s: (i, 0)),
        pl.BlockSpec((_ROWS, 64), lambda i, s: (s * _NB + i, 0)),
        pl.BlockSpec((_ROWS, 16), lambda i, s: (i, 0)),
        pl.BlockSpec((_ROWS, 16), lambda i, s: (i, 0)),
    ]
    return pl.pallas_call(
        functools.partial(_apply_proj_body, nheads=nheads),
        grid=(_NB, 8),
        in_specs=u_specs + other_specs,
        out_specs=out_specs,
        out_shape=outs,
        scratch_shapes=[pltpu.VMEM((_ROWS, 512), jnp.float32)],
    )(uf, uf, uf, uf, uf, uf, uf, uf, den, den, sums, g2, b2, res, Wcat8, Wsd)


def _apply_final_body(u0, u1, u2, u3, u4, u5, u6, u7, d0_ref, d1_ref,
                      sums_ref, g_ref, b_ref, res_ref, y_ref, *, nheads):
    y_ref[...] = _y_block((u0, u1, u2, u3, u4, u5, u6, u7), d0_ref, d1_ref,
                          sums_ref, g_ref, b_ref, res_ref, nheads)


def _apply_final(uf, den, sums, g2, b2, res, nheads):
    u_specs = [
        pl.BlockSpec((_ROWS, 64), lambda i, k=k: (k * _NB + i, 0))
        for k in range(8)
    ]
    other_specs = [
        pl.BlockSpec((_ROWS, 16), lambda i: (i, 0)),
        pl.BlockSpec((_ROWS, 16), lambda i: (_NB + i, 0)),
        pl.BlockSpec((2, 512), lambda i: (0, 0)),
        pl.BlockSpec((1, 512), lambda i: (0, 0)),
        pl.BlockSpec((1, 512), lambda i: (0, 0)),
        pl.BlockSpec((_ROWS, 512), lambda i: (i, 0)),
    ]
    return pl.pallas_call(
        functools.partial(_apply_final_body, nheads=nheads),
        grid=(_NB,),
        in_specs=u_specs + other_specs,
        out_specs=pl.BlockSpec((_ROWS, 512), lambda i: (i, 0)),
        out_shape=jax.ShapeDtypeStruct((_N, 512), jnp.float32),
    )(uf, uf, uf, uf, uf, uf, uf, uf, den, den, sums, g2, b2, res)


def _readout_body(y_ref, watt_ref, batt_ref, o_ref):
    i = pl.program_id(0)

    @pl.when(i == 0)
    def _():
        o_ref[...] = jnp.zeros_like(o_ref)

    y = y_ref[...]
    l = jnp.dot(y, watt_ref[...], preferred_element_type=jnp.float32) \
        + batt_ref[...]
    m = jnp.max(l, axis=1, keepdims=True)
    ex = jnp.exp(l - m)
    att = ex / jnp.sum(ex, axis=1, keepdims=True)
    rc = jnp.sum(att[:, 0:1] * y, axis=0, keepdims=True)
    ro = jnp.sum(att[:, 1:2] * y, axis=0, keepdims=True)
    o_ref[...] += jnp.concatenate([rc, ro], axis=0)


def _readout(y, Watt, batt2):
    return pl.pallas_call(
        _readout_body,
        grid=(_NB,),
        in_specs=[
            pl.BlockSpec((_ROWS, 512), lambda i: (i, 0)),
            pl.BlockSpec((512, 2), lambda i: (0, 0)),
            pl.BlockSpec((1, 2), lambda i: (0, 0)),
        ],
        out_specs=pl.BlockSpec((2, 512), lambda i: (0, 0)),
        out_shape=jax.ShapeDtypeStruct((2, 512), jnp.float32),
    )(y, Watt, batt2)


def _heads_body(r_ref, wc_ref, bc_ref, wo_ref, bo_ref, wco_ref, bco_ref,
                xc_ref, xo_ref, xco_ref):
    rc = r_ref[0:1] * (1.0 / _N)
    ro = r_ref[1:2] * (1.0 / _N)
    xc_ref[...] = jnp.dot(rc, wc_ref[...],
                          preferred_element_type=jnp.float32) + bc_ref[...]
    xo_ref[...] = jnp.dot(ro, wo_ref[...],
                          preferred_element_type=jnp.float32) + bo_ref[...]
    xco_ref[...] = jnp.dot(rc + ro, wco_ref[...],
                           preferred_element_type=jnp.float32) + bco_ref[...]


def _heads(rsums, Wc, bc2, Wo, bo2, Wco, bco2):
    o = jax.ShapeDtypeStruct((1, 10), jnp.float32)
    return pl.pallas_call(
        _heads_body,
        out_shape=[o, o, o],
    )(rsums, Wc, bc2, Wo, bo2, Wco, bco2)


def _phase1_body(src_ref, dst_ref, ssrc_ref, sdst_ref, wexp_ref, den_ref,
                 isA, idA, raA, rbA, isB, idB, raB, rbB,
                 wbuf, zbuf, den_sh, semiA, semgA, semiB, semgB, *, nheads):
    c = lax.axis_index("c")
    s = lax.axis_index("s")
    lane = lax.iota(jnp.int32, 16)

    def zfill(i, carry):
        zbuf[i] = jnp.zeros((16,), jnp.float32)
        return carry

    lax.fori_loop(0, 208, zfill, 0)
    for j in range(3):
        pltpu.sync_copy(zbuf, den_sh.at[pl.ds(s * _RPT + j * 208, 208)])

    @pl.when(s == 0)
    def _():
        pltpu.sync_copy(zbuf.at[pl.ds(0, 16)], den_sh.at[pl.ds(16 * _RPT, 16)])

    plsc.subcore_barrier()

    trips = 39 + jnp.where(s == 0, 1, 0)

    def stage1(b, idx_s, idx_d, semi):
        base = c * (_E // 2) + (b * 16 + s) * 128
        pltpu.async_copy(src_ref.at[pl.ds(base, 128)], idx_s, semi)
        pltpu.async_copy(dst_ref.at[pl.ds(base, 128)], idx_d, semi)

    def wait_stage1(idx_s, idx_d, semi):
        pltpu.make_async_copy(src_ref.at[pl.ds(0, 128)], idx_s, semi).wait()
        pltpu.make_async_copy(dst_ref.at[pl.ds(0, 128)], idx_d, semi).wait()

    def fire_gather(idx_s, idx_d, rows_a, rows_b, semg):
        pltpu.async_copy(ssrc_ref.at[idx_s], rows_a, semg)
        pltpu.async_copy(sdst_ref.at[idx_d], rows_b, semg)

    def wait_gather(rows_a, rows_b, semg):
        pltpu.make_async_copy(ssrc_ref.at[pl.ds(0, 128)], rows_a, semg).wait()
        pltpu.make_async_copy(sdst_ref.at[pl.ds(0, 128)], rows_b, semg).wait()

    stage1(0, isA, idA, semiA)
    wait_stage1(isA, idA, semiA)
    fire_gather(isA, idA, raA, rbA, semgA)

    def process(b, idx_s, idx_d, rows_a, rows_b, semi, semg,
                nidx_s, nidx_d, nrows_a, nrows_b, nsemi, nsemg):
        base = c * (_E // 2) + (b * 16 + s) * 128

        @pl.when(b + 1 < trips)
        def _():
            stage1(b + 1, nidx_s, nidx_d, nsemi)

        wait_gather(rows_a, rows_b, semg)

        def edge(k, ec):
            t = rows_a[k] + rows_b[k]
            l = jnp.where(t >= 0, t, 0.2 * t)
            wbuf[k] = jnp.where(lane < nheads, jnp.exp(l), 0.0)
            return ec

        lax.fori_loop(0, 128, edge, 0)

        @pl.when(b + 1 < trips)
        def _():
            wait_stage1(nidx_s, nidx_d, nsemi)
            fire_gather(nidx_s, nidx_d, nrows_a, nrows_b, nsemg)

        pltpu.sync_copy(wbuf, wexp_ref.at[pl.ds(base, 128)])
        pltpu.sync_copy(wbuf, den_sh.at[idx_d], add=True)

    def piter(b, carry):
        @pl.when(b % 2 == 0)
        def _():
            process(b, isA, idA, raA, rbA, semiA, semgA,
                    isB, idB, raB, rbB, semiB, semgB)

        @pl.when(b % 2 == 1)
        def _():
            process(b, isB, idB, raB, rbB, semiB, semgB,
                    isA, idA, raA, rbA, semiA, semgA)

        return carry

    lax.fori_loop(0, trips, piter, 0)
    plsc.subcore_barrier()
    pltpu.sync_copy(den_sh.at[pl.ds(s * _RPT, _RPT)],
                    den_ref.at[pl.ds(c * _N + s * _RPT, _RPT)])

    @pl.when(s == 0)
    def _():
        pltpu.sync_copy(den_sh.at[pl.ds(16 * _RPT, 16)],
                        den_ref.at[pl.ds(c * _N + 16 * _RPT, 16)])


def _sc_phase1(src, dst, ss16, sd16, nheads):
    body = functools.partial(_phase1_body, nheads=nheads)
    f = pl.kernel(
        body,
        out_type=[
            jax.ShapeDtypeStruct((_E, 16), jnp.float32),
            jax.ShapeDtypeStruct((2 * _N, 16), jnp.float32),
        ],
        mesh=plsc.VectorSubcoreMesh(core_axis_name="c", subcore_axis_name="s"),
        compiler_params=pltpu.CompilerParams(use_tc_tiling_on_sc=False, needs_layout_passes=False),
        scratch_types=[
            pltpu.VMEM((128,), jnp.int32),
            pltpu.VMEM((128,), jnp.int32),
            pltpu.VMEM((128, 16), jnp.float32),
            pltpu.VMEM((128, 16), jnp.float32),
            pltpu.VMEM((128,), jnp.int32),
            pltpu.VMEM((128,), jnp.int32),
            pltpu.VMEM((128, 16), jnp.float32),
            pltpu.VMEM((128, 16), jnp.float32),
            pltpu.VMEM((128, 16), jnp.float32),
            pltpu.VMEM((208, 16), jnp.float32),
            pltpu.VMEM_SHARED((_N, 16), jnp.float32),
            pltpu.SemaphoreType.DMA,
            pltpu.SemaphoreType.DMA,
            pltpu.SemaphoreType.DMA,
            pltpu.SemaphoreType.DMA,
        ],
    )
    return f(src, dst, ss16, sd16)


def _lane_splat(v, h):
    idxv = jnp.full((16,), h, jnp.int32)
    return lax.gather(
        v, idxv[:, None],
        lax.GatherDimensionNumbers(offset_dims=(), collapsed_slice_dims=(0,),
                                   start_index_map=(0,)),
        slice_sizes=(1,), mode=lax.GatherScatterMode.PROMISE_IN_BOUNDS)


def _phase2_body(src_ref, dst_ref, wexp_ref, zf_ref, u_ref,
                 isA, idA, wrA, zrA, isB, idB, wrB, zrB,
                 obuf, zbuf, acc_sh, semiA, semgA, semiB, semgB, *, nheads):
    c = lax.axis_index("c")
    s = lax.axis_index("s")

    def zfill(i, carry):
        for q in range(4):
            zbuf[i, pl.ds(q * 16, 16)] = jnp.zeros((16,), jnp.float32)
        return carry

    trips = 39 + jnp.where(s == 0, 1, 0)

    def stage1(b, idx_s, idx_d, wrows, semi):
        base = (b * 16 + s) * 256
        for j in range(2):
            pltpu.async_copy(src_ref.at[pl.ds(base + j * 128, 128)],
                             idx_s.at[j], semi)
            pltpu.async_copy(dst_ref.at[pl.ds(base + j * 128, 128)],
                             idx_d.at[j], semi)
        pltpu.async_copy(wexp_ref.at[pl.ds(base, 256)], wrows, semi)

    def wait_stage1(idx_s, idx_d, wrows, semi):
        for j in range(2):
            pltpu.make_async_copy(src_ref.at[pl.ds(0, 128)],
                                  idx_s.at[j], semi).wait()
            pltpu.make_async_copy(dst_ref.at[pl.ds(0, 128)],
                                  idx_d.at[j], semi).wait()
        pltpu.make_async_copy(wexp_ref.at[pl.ds(0, 256)], wrows, semi).wait()

    def fire_gather(idx_s, zrows, semg, off):
        for j in range(2):
            for q in range(8):
                idx_s[j, pl.ds(q * 16, 16)] = \
                    idx_s[j, pl.ds(q * 16, 16)] + off
        for j in range(2):
            pltpu.async_copy(zf_ref.at[idx_s.at[j]],
                             zrows.at[pl.ds(j * 128, 128)], semg)

    def wait_gather(zrows, semg):
        for j in range(2):
            pltpu.make_async_copy(zf_ref.at[pl.ds(0, 128)],
                                  zrows.at[pl.ds(j * 128, 128)], semg).wait()

    for sl in range(4):
        slice_id = c * 4 + sl
        off = slice_id * _N
        lax.fori_loop(0, 208, zfill, 0)
        for j in range(3):
            pltpu.sync_copy(zbuf, acc_sh.at[pl.ds(s * _RPT + j * 208, 208)])

        @pl.when(s == 0)
        def _():
            pltpu.sync_copy(zbuf.at[pl.ds(0, 16)],
                            acc_sh.at[pl.ds(16 * _RPT, 16)])

        plsc.subcore_barrier()

        stage1(0, isA, idA, wrA, semiA)
        wait_stage1(isA, idA, wrA, semiA)
        fire_gather(isA, zrA, semgA, off)

        def process(b, idx_s, idx_d, wrows, zrows, semi, semg,
                    nidx_s, nidx_d, nwrows, nzrows, nsemi, nsemg):
            @pl.when(b + 1 < trips)
            def _():
                stage1(b + 1, nidx_s, nidx_d, nwrows, nsemi)

            wait_gather(zrows, semg)

            def edge(k, ec):
                wrow = wrows[k]
                if nheads == 8:
                    scv = _lane_splat(wrow, slice_id)
                else:
                    scv = _lane_splat(wrow, 0)
                for q in range(4):
                    obuf[k, pl.ds(q * 16, 16)] = \
                        zrows[k, pl.ds(q * 16, 16)] * scv
                return ec

            lax.fori_loop(0, 256, edge, 0)

            @pl.when(b + 1 < trips)
            def _():
                wait_stage1(nidx_s, nidx_d, nwrows, nsemi)
                fire_gather(nidx_s, nzrows, nsemg, off)

            for j in range(2):
                pltpu.sync_copy(obuf.at[pl.ds(j * 128, 128)],
                                acc_sh.at[idx_d.at[j]], add=True)

        def piter(b, carry):
            @pl.when(b % 2 == 0)
            def _():
                process(b, isA, idA, wrA, zrA, semiA, semgA,
                        isB, idB, wrB, zrB, semiB, semgB)

            @pl.when(b % 2 == 1)
            def _():
                process(b, isB, idB, wrB, zrB, semiB, semgB,
                        isA, idA, wrA, zrA, semiA, semgA)

            return carry

        lax.fori_loop(0, trips, piter, 0)
        plsc.subcore_barrier()
        pltpu.sync_copy(acc_sh.at[pl.ds(s * _RPT, _RPT)],
                        u_ref.at[pl.ds(slice_id * _N + s * _RPT, _RPT)])

        @pl.when(s == 0)
        def _():
            pltpu.sync_copy(acc_sh.at[pl.ds(16 * _RPT, 16)],
                            u_ref.at[pl.ds(slice_id * _N + 16 * _RPT, 16)])

        if sl < 3:
            plsc.subcore_barrier()


def _sc_phase2(src, dst, wexp, zf, nheads):
    body = functools.partial(_phase2_body, nheads=nheads)
    f = pl.kernel(
        body,
        out_type=jax.ShapeDtypeStruct((8 * _N, 64), jnp.float32),
        mesh=plsc.VectorSubcoreMesh(core_axis_name="c", subcore_axis_name="s"),
        compiler_params=pltpu.CompilerParams(use_tc_tiling_on_sc=False, needs_layout_passes=False),
        scratch_types=[
            pltpu.VMEM((2, 128), jnp.int32),
            pltpu.VMEM((2, 128), jnp.int32),
            pltpu.VMEM((256, 16), jnp.float32),
            pltpu.VMEM((256, 64), jnp.float32),
            pltpu.VMEM((2, 128), jnp.int32),
            pltpu.VMEM((2, 128), jnp.int32),
            pltpu.VMEM((256, 16), jnp.float32),
            pltpu.VMEM((256, 64), jnp.float32),
            pltpu.VMEM((256, 64), jnp.float32),
            pltpu.VMEM((208, 64), jnp.float32),
            pltpu.VMEM_SHARED((_N, 64), jnp.float32),
            pltpu.SemaphoreType.DMA,
            pltpu.SemaphoreType.DMA,
            pltpu.SemaphoreType.DMA,
            pltpu.SemaphoreType.DMA,
        ],
    )
    return f(src, dst, wexp, zf)


def _blockdiag_attn(a, nheads, hid):
    out = jnp.zeros((nheads * hid, 16), jnp.float32)
    for h in range(nheads):
        out = out.at[h * hid:(h + 1) * hid, h].set(a[h])
    return out


def kernel(h, edge_index, e, W_emb, b_emb, W0, a0, g0, bta0, W1, a1, g1,
           bta1, Wf, af, gf, btf, Watt, batt, Wc, bc, Wo, bo, Wco, bco):
    src = edge_index[0]
    dst = edge_index[1]

    layers = []
    for (W, a, g, bta) in ((W0, a0, g0, bta0), (W1, a1, g1, bta1),
                           (Wf, af, gf, btf)):
        H, d, hid = W.shape
        Wcat = jnp.transpose(W, (1, 0, 2)).reshape(d, H * hid)
        Asrc = _blockdiag_attn(a[:, :hid], H, hid)
        Adst = _blockdiag_attn(a[:, hid:], H, hid)
        Wsd = jnp.concatenate([Wcat @ Asrc, Wcat @ Adst], axis=1)
        Wcat8 = jnp.transpose(Wcat.reshape(d, 8, 64), (1, 0, 2))
        layers.append((H, Wcat8, Wsd, g.reshape(1, -1), bta.reshape(1, -1)))

    x = _emb_matmul(h, W_emb, b_emb.reshape(1, -1))

    H0, Wcat80, Wsd0, _, _ = layers[0]
    zf, ss16, sd16 = _proj(x, Wcat80, Wsd0)

    res = x
    y = None
    for li in range(3):
        H, _, Wsd, g2, b2 = layers[li]
        wexp, den = _sc_phase1(src, dst, ss16, sd16, H)
        uf = _sc_phase2(src, dst, wexp, zf, H)
        sums = _stats(uf, den, H)
        if li < 2:
            Hn, Wcat8n, Wsdn, _, _ = layers[li + 1]
            y, zf, ss16, sd16 = _apply_proj(uf, den, sums, g2, b2, res,
                                            Wcat8n, Wsdn, H)
        else:
            y = _apply_final(uf, den, sums, g2, b2, res, H)
        res = y

    rsums = _readout(y, Watt, batt.reshape(1, -1))
    xc, xo, xco = _heads(rsums, Wc, bc.reshape(1, -1), Wo, bo.reshape(1, -1),
                         Wco, bco.reshape(1, -1))
    return (xc, xo, xco)


# speedup vs baseline: 27.6365x; 1.5054x over previous
import functools

import jax
import jax.numpy as jnp
from jax import lax
from jax.experimental import pallas as pl
from jax.experimental.pallas import tpu as pltpu
from jax.experimental.pallas import tpu_sc as plsc

_N = 10000
_E = 160000
_ROWS = 1000
_NB = _N // _ROWS
_RPT = 624


def _mm_bias_body(x_ref, w_ref, b_ref, o_ref):
    o_ref[...] = jnp.dot(x_ref[...], w_ref[...],
                         preferred_element_type=jnp.float32) + b_ref[...]


def _emb_matmul(h, W, b2):
    k = h.shape[1]
    m = W.shape[1]
    return pl.pallas_call(
        _mm_bias_body,
        grid=(_NB,),
        in_specs=[
            pl.BlockSpec((_ROWS, k), lambda i: (i, 0)),
            pl.BlockSpec((k, m), lambda i: (0, 0)),
            pl.BlockSpec((1, m), lambda i: (0, 0)),
        ],
        out_specs=pl.BlockSpec((_ROWS, m), lambda i: (i, 0)),
        out_shape=jax.ShapeDtypeStruct((_N, m), jnp.float32),
    )(h, W, b2)


def _proj_body(x_ref, wcat_ref, wsd_ref, z_ref, ss_ref, sd_ref):
    x = x_ref[...]
    z_ref[...] = jnp.dot(x, wcat_ref[0], preferred_element_type=jnp.float32)
    s = jnp.dot(x, wsd_ref[...], preferred_element_type=jnp.float32)
    ss_ref[...] = s[:, 0:16]
    sd_ref[...] = s[:, 16:32]


def _proj(x, Wcat8, Wsd):
    outs = [
        jax.ShapeDtypeStruct((8 * _N, 64), jnp.float32),
        jax.ShapeDtypeStruct((_N, 16), jnp.float32),
        jax.ShapeDtypeStruct((_N, 16), jnp.float32),
    ]
    out_specs = [
        pl.BlockSpec((_ROWS, 64), lambda i, s: (s * _NB + i, 0)),
        pl.BlockSpec((_ROWS, 16), lambda i, s: (i, 0)),
        pl.BlockSpec((_ROWS, 16), lambda i, s: (i, 0)),
    ]
    return pl.pallas_call(
        _proj_body,
        grid=(_NB, 8),
        in_specs=[
            pl.BlockSpec((_ROWS, 512), lambda i, s: (i, 0)),
            pl.BlockSpec((1, 512, 64), lambda i, s: (s, 0, 0)),
            pl.BlockSpec((512, 32), lambda i, s: (0, 0)),
        ],
        out_specs=out_specs,
        out_shape=outs,
    )(x, Wcat8, Wsd)


def _t_block(u_refs, d0_ref, d1_ref, nheads):
    d = d0_ref[...] + d1_ref[...]
    cols = []
    for j, u in enumerate(u_refs):
        h = j if nheads == 8 else 0
        cols.append(u[...] / (d[:, h:h + 1] + 1e-9))
    return jnp.concatenate(cols, axis=1)


def _stats_body(u0, u1, u2, u3, u4, u5, u6, u7, d0_ref, d1_ref, o_ref, *,
                nheads):
    i = pl.program_id(0)

    @pl.when(i == 0)
    def _():
        o_ref[...] = jnp.zeros_like(o_ref)

    t = _t_block((u0, u1, u2, u3, u4, u5, u6, u7), d0_ref, d1_ref, nheads)
    s = jnp.sum(t, axis=0, keepdims=True)
    ss = jnp.sum(t * t, axis=0, keepdims=True)
    o_ref[...] += jnp.concatenate([s, ss], axis=0)


def _stats(uf, den, nheads):
    u_specs = [
        pl.BlockSpec((_ROWS, 64), lambda i, k=k: (k * _NB + i, 0))
        for k in range(8)
    ]
    d_specs = [
        pl.BlockSpec((_ROWS, 16), lambda i: (i, 0)),
        pl.BlockSpec((_ROWS, 16), lambda i: (_NB + i, 0)),
    ]
    return pl.pallas_call(
        functools.partial(_stats_body, nheads=nheads),
        grid=(_NB,),
        in_specs=u_specs + d_specs,
        out_specs=pl.BlockSpec((2, 512), lambda i: (0, 0)),
        out_shape=jax.ShapeDtypeStruct((2, 512), jnp.float32),
    )(uf, uf, uf, uf, uf, uf, uf, uf, den, den)


def _y_block(u_refs, d0_ref, d1_ref, sums_ref, g_ref, b_ref, res_ref, nheads):
    t = _t_block(u_refs, d0_ref, d1_ref, nheads)
    sums = sums_ref[...]
    mean = sums[0:1] * (1.0 / _N)
    var = sums[1:2] * (1.0 / _N) - mean * mean
    rstd = lax.rsqrt(var + 1e-5)
    yv = (t - mean) * rstd * g_ref[...] + b_ref[...]
    yv = jnp.where(yv > 0, yv, jnp.exp(jnp.minimum(yv, 0.0)) - 1.0)
    return yv + res_ref[...]


def _apply_proj_body(u0, u1, u2, u3, u4, u5, u6, u7, d0_ref, d1_ref,
                     sums_ref, g_ref, b_ref, res_ref, wcat_ref, wsd_ref,
                     y_ref, z_ref, ss_ref, sd_ref, y_scr, *, nheads):
    s = pl.program_id(1)

    @pl.when(s == 0)
    def _():
        yv = _y_block((u0, u1, u2, u3, u4, u5, u6, u7), d0_ref, d1_ref,
                      sums_ref, g_ref, b_ref, res_ref, nheads)
        y_scr[...] = yv
        y_ref[...] = yv
        sv = jnp.dot(yv, wsd_ref[...], preferred_element_type=jnp.float32)
        ss_ref[...] = sv[:, 0:16]
        sd_ref[...] = sv[:, 16:32]

    z_ref[...] = jnp.dot(y_scr[...], wcat_ref[0],
                         preferred_element_type=jnp.float32)


def _apply_proj(uf, den, sums, g2, b2, res, Wcat8, Wsd, nheads):
    u_specs = [
        pl.BlockSpec((_ROWS, 64), lambda i, s, k=k: (k * _NB + i, 0))
        for k in range(8)
    ]
    other_specs = [
        pl.BlockSpec((_ROWS, 16), lambda i, s: (i, 0)),
        pl.BlockSpec((_ROWS, 16), lambda i, s: (_NB + i, 0)),
        pl.BlockSpec((2, 512), lambda i, s: (0, 0)),
        pl.BlockSpec((1, 512), lambda i, s: (0, 0)),
        pl.BlockSpec((1, 512), lambda i, s: (0, 0)),
        pl.BlockSpec((_ROWS, 512), lambda i, s: (i, 0)),
        pl.BlockSpec((1, 512, 64), lambda i, s: (s, 0, 0)),
        pl.BlockSpec((512, 32), lambda i, s: (0, 0)),
    ]
    outs = [
        jax.ShapeDtypeStruct((_N, 512), jnp.float32),
        jax.ShapeDtypeStruct((8 * _N, 64), jnp.float32),
        jax.ShapeDtypeStruct((_N, 16), jnp.float32),
        jax.ShapeDtypeStruct((_N, 16), jnp.float32),
    ]
    out_specs = [
        pl.BlockSpec((_ROWS, 512), lambda i, s: (i, 0)),
        pl.BlockSpec((_ROWS, 64), lambda i, s: (s * _NB + i, 0)),
        pl.BlockSpec((_ROWS, 16), lambda i, s: (i, 0)),
        pl.BlockSpec((_ROWS, 16), lambda i, s: (i, 0)),
    ]
    return pl.pallas_call(
        functools.partial(_apply_proj_body, nheads=nheads),
        grid=(_NB, 8),
        in_specs=u_specs + other_specs,
        out_specs=out_specs,
        out_shape=outs,
        scratch_shapes=[pltpu.VMEM((_ROWS, 512), jnp.float32)],
    )(uf, uf, uf, uf, uf, uf, uf, uf, den, den, sums, g2, b2, res, Wcat8, Wsd)


def _apply_final_body(u0, u1, u2, u3, u4, u5, u6, u7, d0_ref, d1_ref,
                      sums_ref, g_ref, b_ref, res_ref, y_ref, *, nheads):
    y_ref[...] = _y_block((u0, u1, u2, u3, u4, u5, u6, u7), d0_ref, d1_ref,
                          sums_ref, g_ref, b_ref, res_ref, nheads)


def _apply_final(uf, den, sums, g2, b2, res, nheads):
    u_specs = [
        pl.BlockSpec((_ROWS, 64), lambda i, k=k: (k * _NB + i, 0))
        for k in range(8)
    ]
    other_specs = [
        pl.BlockSpec((_ROWS, 16), lambda i: (i, 0)),
        pl.BlockSpec((_ROWS, 16), lambda i: (_NB + i, 0)),
        pl.BlockSpec((2, 512), lambda i: (0, 0)),
        pl.BlockSpec((1, 512), lambda i: (0, 0)),
        pl.BlockSpec((1, 512), lambda i: (0, 0)),
        pl.BlockSpec((_ROWS, 512), lambda i: (i, 0)),
    ]
    return pl.pallas_call(
        functools.partial(_apply_final_body, nheads=nheads),
        grid=(_NB,),
        in_specs=u_specs + other_specs,
        out_specs=pl.BlockSpec((_ROWS, 512), lambda i: (i, 0)),
        out_shape=jax.ShapeDtypeStruct((_N, 512), jnp.float32),
    )(uf, uf, uf, uf, uf, uf, uf, uf, den, den, sums, g2, b2, res)


def _readout_body(y_ref, watt_ref, batt_ref, o_ref):
    i = pl.program_id(0)

    @pl.when(i == 0)
    def _():
        o_ref[...] = jnp.zeros_like(o_ref)

    y = y_ref[...]
    l = jnp.dot(y, watt_ref[...], preferred_element_type=jnp.float32) \
        + batt_ref[...]
    m = jnp.max(l, axis=1, keepdims=True)
    ex = jnp.exp(l - m)
    att = ex / jnp.sum(ex, axis=1, keepdims=True)
    rc = jnp.sum(att[:, 0:1] * y, axis=0, keepdims=True)
    ro = jnp.sum(att[:, 1:2] * y, axis=0, keepdims=True)
    o_ref[...] += jnp.concatenate([rc, ro], axis=0)


def _readout(y, Watt, batt2):
    return pl.pallas_call(
        _readout_body,
        grid=(_NB,),
        in_specs=[
            pl.BlockSpec((_ROWS, 512), lambda i: (i, 0)),
            pl.BlockSpec((512, 2), lambda i: (0, 0)),
            pl.BlockSpec((1, 2), lambda i: (0, 0)),
        ],
        out_specs=pl.BlockSpec((2, 512), lambda i: (0, 0)),
        out_shape=jax.ShapeDtypeStruct((2, 512), jnp.float32),
    )(y, Watt, batt2)


def _heads_body(r_ref, wc_ref, bc_ref, wo_ref, bo_ref, wco_ref, bco_ref,
                xc_ref, xo_ref, xco_ref):
    rc = r_ref[0:1] * (1.0 / _N)
    ro = r_ref[1:2] * (1.0 / _N)
    xc_ref[...] = jnp.dot(rc, wc_ref[...],
                          preferred_element_type=jnp.float32) + bc_ref[...]
    xo_ref[...] = jnp.dot(ro, wo_ref[...],
                          preferred_element_type=jnp.float32) + bo_ref[...]
    xco_ref[...] = jnp.dot(rc + ro, wco_ref[...],
                           preferred_element_type=jnp.float32) + bco_ref[...]


def _heads(rsums, Wc, bc2, Wo, bo2, Wco, bco2):
    o = jax.ShapeDtypeStruct((1, 10), jnp.float32)
    return pl.pallas_call(
        _heads_body,
        out_shape=[o, o, o],
    )(rsums, Wc, bc2, Wo, bo2, Wco, bco2)


def _phase1_body(src_ref, dst_ref, ssrc_ref, sdst_ref, wexp_ref, den_ref,
                 isA, idA, raA, rbA, isB, idB, raB, rbB,
                 wbuf, zbuf, den_sh, semiA, semgA, semiB, semgB, *, nheads):
    c = lax.axis_index("c")
    s = lax.axis_index("s")
    lane = lax.iota(jnp.int32, 16)

    def zfill(i, carry):
        zbuf[i] = jnp.zeros((16,), jnp.float32)
        return carry

    lax.fori_loop(0, 208, zfill, 0)
    for j in range(3):
        pltpu.sync_copy(zbuf, den_sh.at[pl.ds(s * _RPT + j * 208, 208)])

    @pl.when(s == 0)
    def _():
        pltpu.sync_copy(zbuf.at[pl.ds(0, 16)], den_sh.at[pl.ds(16 * _RPT, 16)])

    plsc.subcore_barrier()

    trips = 39 + jnp.where(s == 0, 1, 0)

    def stage1(b, idx_s, idx_d, semi):
        base = c * (_E // 2) + (b * 16 + s) * 128
        pltpu.async_copy(src_ref.at[pl.ds(base, 128)], idx_s, semi)
        pltpu.async_copy(dst_ref.at[pl.ds(base, 128)], idx_d, semi)

    def wait_stage1(idx_s, idx_d, semi):
        pltpu.make_async_copy(src_ref.at[pl.ds(0, 128)], idx_s, semi).wait()
        pltpu.make_async_copy(dst_ref.at[pl.ds(0, 128)], idx_d, semi).wait()

    def fire_gather(idx_s, idx_d, rows_a, rows_b, semg):
        pltpu.async_copy(ssrc_ref.at[idx_s], rows_a, semg)
        pltpu.async_copy(sdst_ref.at[idx_d], rows_b, semg)

    def wait_gather(rows_a, rows_b, semg):
        pltpu.make_async_copy(ssrc_ref.at[pl.ds(0, 128)], rows_a, semg).wait()
        pltpu.make_async_copy(sdst_ref.at[pl.ds(0, 128)], rows_b, semg).wait()

    stage1(0, isA, idA, semiA)
    wait_stage1(isA, idA, semiA)
    fire_gather(isA, idA, raA, rbA, semgA)

    def process(b, idx_s, idx_d, rows_a, rows_b, semi, semg,
                nidx_s, nidx_d, nrows_a, nrows_b, nsemi, nsemg):
        base = c * (_E // 2) + (b * 16 + s) * 128

        @pl.when(b + 1 < trips)
        def _():
            stage1(b + 1, nidx_s, nidx_d, nsemi)

        wait_gather(rows_a, rows_b, semg)

        @functools.partial(plsc.parallel_loop, 0, 128, unroll=8)
        def _(k):
            t = rows_a[k] + rows_b[k]
            l = jnp.where(t >= 0, t, 0.2 * t)
            wbuf[k] = jnp.where(lane < nheads, jnp.exp(l), 0.0)

        @pl.when(b + 1 < trips)
        def _():
            wait_stage1(nidx_s, nidx_d, nsemi)
            fire_gather(nidx_s, nidx_d, nrows_a, nrows_b, nsemg)

        pltpu.sync_copy(wbuf, wexp_ref.at[pl.ds(base, 128)])
        pltpu.sync_copy(wbuf, den_sh.at[idx_d], add=True)

    def piter(b, carry):
        @pl.when(b % 2 == 0)
        def _():
            process(b, isA, idA, raA, rbA, semiA, semgA,
                    isB, idB, raB, rbB, semiB, semgB)

        @pl.when(b % 2 == 1)
        def _():
            process(b, isB, idB, raB, rbB, semiB, semgB,
                    isA, idA, raA, rbA, semiA, semgA)

        return carry

    lax.fori_loop(0, trips, piter, 0)
    plsc.subcore_barrier()
    pltpu.sync_copy(den_sh.at[pl.ds(s * _RPT, _RPT)],
                    den_ref.at[pl.ds(c * _N + s * _RPT, _RPT)])

    @pl.when(s == 0)
    def _():
        pltpu.sync_copy(den_sh.at[pl.ds(16 * _RPT, 16)],
                        den_ref.at[pl.ds(c * _N + 16 * _RPT, 16)])


def _sc_phase1(src, dst, ss16, sd16, nheads):
    body = functools.partial(_phase1_body, nheads=nheads)
    f = pl.kernel(
        body,
        out_type=[
            jax.ShapeDtypeStruct((_E, 16), jnp.float32),
            jax.ShapeDtypeStruct((2 * _N, 16), jnp.float32),
        ],
        mesh=plsc.VectorSubcoreMesh(core_axis_name="c", subcore_axis_name="s"),
        compiler_params=pltpu.CompilerParams(use_tc_tiling_on_sc=False, needs_layout_passes=False),
        scratch_types=[
            pltpu.VMEM((128,), jnp.int32),
            pltpu.VMEM((128,), jnp.int32),
            pltpu.VMEM((128, 16), jnp.float32),
            pltpu.VMEM((128, 16), jnp.float32),
            pltpu.VMEM((128,), jnp.int32),
            pltpu.VMEM((128,), jnp.int32),
            pltpu.VMEM((128, 16), jnp.float32),
            pltpu.VMEM((128, 16), jnp.float32),
            pltpu.VMEM((128, 16), jnp.float32),
            pltpu.VMEM((208, 16), jnp.float32),
            pltpu.VMEM_SHARED((_N, 16), jnp.float32),
            pltpu.SemaphoreType.DMA,
            pltpu.SemaphoreType.DMA,
            pltpu.SemaphoreType.DMA,
            pltpu.SemaphoreType.DMA,
        ],
    )
    return f(src, dst, ss16, sd16)


def _lane_splat(v, h):
    idxv = jnp.full((16,), h, jnp.int32)
    return lax.gather(
        v, idxv[:, None],
        lax.GatherDimensionNumbers(offset_dims=(), collapsed_slice_dims=(0,),
                                   start_index_map=(0,)),
        slice_sizes=(1,), mode=lax.GatherScatterMode.PROMISE_IN_BOUNDS)


def _phase2_body(src_ref, dst_ref, wexp_ref, zf_ref, u_ref,
                 isA, idA, wrA, zrA, isB, idB, wrB, zrB,
                 obufA, obufB, zbuf, acc_sh,
                 semiA, semgA, semsA, semiB, semgB, semsB, *, nheads):
    c = lax.axis_index("c")
    s = lax.axis_index("s")

    def zfill(i, carry):
        for q in range(4):
            zbuf[i, pl.ds(q * 16, 16)] = jnp.zeros((16,), jnp.float32)
        return carry

    trips = 39 + jnp.where(s == 0, 1, 0)

    def stage1(b, idx_s, idx_d, wrows, semi):
        base = (b * 16 + s) * 256
        for j in range(2):
            pltpu.async_copy(src_ref.at[pl.ds(base + j * 128, 128)],
                             idx_s.at[j], semi)
            pltpu.async_copy(dst_ref.at[pl.ds(base + j * 128, 128)],
                             idx_d.at[j], semi)
        pltpu.async_copy(wexp_ref.at[pl.ds(base, 256)], wrows, semi)

    def wait_stage1(idx_s, idx_d, wrows, semi):
        for j in range(2):
            pltpu.make_async_copy(src_ref.at[pl.ds(0, 128)],
                                  idx_s.at[j], semi).wait()
            pltpu.make_async_copy(dst_ref.at[pl.ds(0, 128)],
                                  idx_d.at[j], semi).wait()
        pltpu.make_async_copy(wexp_ref.at[pl.ds(0, 256)], wrows, semi).wait()

    def fire_gather(idx_s, zrows, semg, off):
        for j in range(2):
            for q in range(8):
                idx_s[j, pl.ds(q * 16, 16)] = \
                    idx_s[j, pl.ds(q * 16, 16)] + off
        for j in range(2):
            pltpu.async_copy(zf_ref.at[idx_s.at[j]],
                             zrows.at[pl.ds(j * 128, 128)], semg)

    def wait_gather(zrows, semg):
        for j in range(2):
            pltpu.make_async_copy(zf_ref.at[pl.ds(0, 128)],
                                  zrows.at[pl.ds(j * 128, 128)], semg).wait()

    for sl in range(4):
        slice_id = c * 4 + sl
        off = slice_id * _N
        lax.fori_loop(0, 208, zfill, 0)
        for j in range(3):
            pltpu.sync_copy(zbuf, acc_sh.at[pl.ds(s * _RPT + j * 208, 208)])

        @pl.when(s == 0)
        def _():
            pltpu.sync_copy(zbuf.at[pl.ds(0, 16)],
                            acc_sh.at[pl.ds(16 * _RPT, 16)])

        plsc.subcore_barrier()

        stage1(0, isA, idA, wrA, semiA)
        wait_stage1(isA, idA, wrA, semiA)
        fire_gather(isA, zrA, semgA, off)

        def wait_scatter(obuf, idx_d, sems):
            for j in range(2):
                pltpu.make_async_copy(obuf.at[pl.ds(j * 128, 128)],
                                      acc_sh.at[idx_d.at[j]], sems).wait()

        def process(b, idx_s, idx_d, wrows, zrows, obuf, semi, semg, sems,
                    nidx_s, nidx_d, nwrows, nzrows, nsemi, nsemg):
            @pl.when(b + 1 < trips)
            def _():
                stage1(b + 1, nidx_s, nidx_d, nwrows, nsemi)

            @pl.when(b >= 2)
            def _():
                wait_scatter(obuf, idx_d, sems)

            wait_gather(zrows, semg)

            @functools.partial(plsc.parallel_loop, 0, 256, unroll=8)
            def _(k):
                wrow = wrows[k]
                if nheads == 8:
                    scv = _lane_splat(wrow, slice_id)
                else:
                    scv = _lane_splat(wrow, 0)
                for q in range(4):
                    obuf[k, pl.ds(q * 16, 16)] = \
                        zrows[k, pl.ds(q * 16, 16)] * scv

            @pl.when(b + 1 < trips)
            def _():
                wait_stage1(nidx_s, nidx_d, nwrows, nsemi)
                fire_gather(nidx_s, nzrows, nsemg, off)

            for j in range(2):
                pltpu.async_copy(obuf.at[pl.ds(j * 128, 128)],
                                 acc_sh.at[idx_d.at[j]], sems, add=True)

        def piter(b, carry):
            @pl.when(b % 2 == 0)
            def _():
                process(b, isA, idA, wrA, zrA, obufA, semiA, semgA, semsA,
                        isB, idB, wrB, zrB, semiB, semgB)

            @pl.when(b % 2 == 1)
            def _():
                process(b, isB, idB, wrB, zrB, obufB, semiB, semgB, semsB,
                        isA, idA, wrA, zrA, semiA, semgA)

            return carry

        lax.fori_loop(0, trips, piter, 0)
        for j in range(2):
            pltpu.make_async_copy(obufA.at[pl.ds(j * 128, 128)],
                                  acc_sh.at[idA.at[j]], semsA).wait()
            pltpu.make_async_copy(obufB.at[pl.ds(j * 128, 128)],
                                  acc_sh.at[idB.at[j]], semsB).wait()
        plsc.subcore_barrier()
        pltpu.sync_copy(acc_sh.at[pl.ds(s * _RPT, _RPT)],
                        u_ref.at[pl.ds(slice_id * _N + s * _RPT, _RPT)])

        @pl.when(s == 0)
        def _():
            pltpu.sync_copy(acc_sh.at[pl.ds(16 * _RPT, 16)],
                            u_ref.at[pl.ds(slice_id * _N + 16 * _RPT, 16)])

        if sl < 3:
            plsc.subcore_barrier()


def _sc_phase2(src, dst, wexp, zf, nheads):
    body = functools.partial(_phase2_body, nheads=nheads)
    f = pl.kernel(
        body,
        out_type=jax.ShapeDtypeStruct((8 * _N, 64), jnp.float32),
        mesh=plsc.VectorSubcoreMesh(core_axis_name="c", subcore_axis_name="s"),
        compiler_params=pltpu.CompilerParams(use_tc_tiling_on_sc=False, needs_layout_passes=False),
        scratch_types=[
            pltpu.VMEM((2, 128), jnp.int32),
            pltpu.VMEM((2, 128), jnp.int32),
            pltpu.VMEM((256, 16), jnp.float32),
            pltpu.VMEM((256, 64), jnp.float32),
            pltpu.VMEM((2, 128), jnp.int32),
            pltpu.VMEM((2, 128), jnp.int32),
            pltpu.VMEM((256, 16), jnp.float32),
            pltpu.VMEM((256, 64), jnp.float32),
            pltpu.VMEM((256, 64), jnp.float32),
            pltpu.VMEM((256, 64), jnp.float32),
            pltpu.VMEM((208, 64), jnp.float32),
            pltpu.VMEM_SHARED((_N, 64), jnp.float32),
            pltpu.SemaphoreType.DMA,
            pltpu.SemaphoreType.DMA,
            pltpu.SemaphoreType.DMA,
            pltpu.SemaphoreType.DMA,
            pltpu.SemaphoreType.DMA,
            pltpu.SemaphoreType.DMA,
        ],
    )
    return f(src, dst, wexp, zf)


def _blockdiag_attn(a, nheads, hid):
    out = jnp.zeros((nheads * hid, 16), jnp.float32)
    for h in range(nheads):
        out = out.at[h * hid:(h + 1) * hid, h].set(a[h])
    return out


def kernel(h, edge_index, e, W_emb, b_emb, W0, a0, g0, bta0, W1, a1, g1,
           bta1, Wf, af, gf, btf, Watt, batt, Wc, bc, Wo, bo, Wco, bco):
    src = edge_index[0]
    dst = edge_index[1]

    layers = []
    for (W, a, g, bta) in ((W0, a0, g0, bta0), (W1, a1, g1, bta1),
                           (Wf, af, gf, btf)):
        H, d, hid = W.shape
        Wcat = jnp.transpose(W, (1, 0, 2)).reshape(d, H * hid)
        Asrc = _blockdiag_attn(a[:, :hid], H, hid)
        Adst = _blockdiag_attn(a[:, hid:], H, hid)
        Wsd = jnp.concatenate([Wcat @ Asrc, Wcat @ Adst], axis=1)
        Wcat8 = jnp.transpose(Wcat.reshape(d, 8, 64), (1, 0, 2))
        layers.append((H, Wcat8, Wsd, g.reshape(1, -1), bta.reshape(1, -1)))

    x = _emb_matmul(h, W_emb, b_emb.reshape(1, -1))

    H0, Wcat80, Wsd0, _, _ = layers[0]
    zf, ss16, sd16 = _proj(x, Wcat80, Wsd0)

    res = x
    y = None
    for li in range(3):
        H, _, Wsd, g2, b2 = layers[li]
        wexp, den = _sc_phase1(src, dst, ss16, sd16, H)
        uf = _sc_phase2(src, dst, wexp, zf, H)
        sums = _stats(uf, den, H)
        if li < 2:
            Hn, Wcat8n, Wsdn, _, _ = layers[li + 1]
            y, zf, ss16, sd16 = _apply_proj(uf, den, sums, g2, b2, res,
                                            Wcat8n, Wsdn, H)
        else:
            y = _apply_final(uf, den, sums, g2, b2, res, H)
        res = y

    rsums = _readout(y, Watt, batt.reshape(1, -1))
    xc, xo, xco = _heads(rsums, Wc, bc.reshape(1, -1), Wo, bo.reshape(1, -1),
                         Wco, bco.reshape(1, -1))
    return (xc, xo, xco)
